# Initial kernel scaffold; baseline (speedup 1.0000x reference)
import jax
import jax.numpy as jnp
from jax.experimental import pallas as pl


def kernel(x, edge_index, batch, conv1_W1, conv1_b1, conv1_gamma, conv1_beta, conv1_W2, conv1_b2, conv2_W1, conv2_b1, conv2_gamma, conv2_beta, conv2_W2, conv2_b2, conv3_W1, conv3_b1, conv3_gamma, conv3_beta, conv3_W2, conv3_b2, lin1_W, lin1_b, lin2_W, lin2_b):
    raise NotImplementedError("write your pallas kernel here")


# speedup vs baseline: 2.5640x; 2.5640x over previous
import functools

import jax
import jax.numpy as jnp
from jax import lax
from jax.experimental import pallas as pl
from jax.experimental.pallas import tpu as pltpu
from jax.experimental.pallas import tpu_sc as plsc

N = 10000
E = 320000
D = 128
G = 64
BN_EPS = 1e-5

NC, NS = 2, 16
NW = NC * NS
CHUNK = 128
CHUNKS_PER_TILE = 80
E_PAD = NW * CHUNKS_PER_TILE * CHUNK
N_PAD = 10240
ROWS_PER_TILE = N_PAD // NS
ZCOPIES = ROWS_PER_TILE // CHUNK

BR = 640
GRID_R = N_PAD // BR


_sc_mesh = plsc.VectorSubcoreMesh(
    core_axis_name="c", subcore_axis_name="s", num_cores=NC, num_subcores=NS)


@functools.partial(
    pl.kernel,
    out_type=jax.ShapeDtypeStruct((NC, N_PAD, D), jnp.float32),
    mesh=_sc_mesh,
    scratch_types=[
        pltpu.VMEM((CHUNKS_PER_TILE, CHUNK), jnp.int32),
        pltpu.VMEM((CHUNKS_PER_TILE, CHUNK), jnp.int32),
        pltpu.VMEM((CHUNK, D), jnp.float32),
        pltpu.VMEM_SHARED((N_PAD, D), jnp.float32),
        pltpu.SemaphoreType.DMA,
    ],
)
def _sc_scatter_add(x_hbm, src_hbm, dst_hbm, zero_hbm, out_hbm,
                    src_v, dst_v, rows_v, acc_sh, sem):
    cid = lax.axis_index("c")
    sid = lax.axis_index("s")
    wid = cid * NS + sid
    row0 = sid * ROWS_PER_TILE

    pltpu.sync_copy(zero_hbm.at[pl.ds(0, CHUNK)], rows_v)
    for k in range(ZCOPIES):
        pltpu.sync_copy(rows_v, acc_sh.at[pl.ds(row0 + k * CHUNK, CHUNK)])
    plsc.subcore_barrier()

    pltpu.sync_copy(src_hbm.at[wid], src_v)
    pltpu.sync_copy(dst_hbm.at[wid], dst_v)

    def body(j, carry):
        pltpu.async_copy(x_hbm.at[src_v.at[j]], rows_v, sem).wait()
        pltpu.sync_copy(rows_v, acc_sh.at[dst_v.at[j]], add=True)
        return carry

    lax.fori_loop(0, CHUNKS_PER_TILE, body, 0)
    plsc.subcore_barrier()

    pltpu.sync_copy(acc_sh.at[pl.ds(row0, ROWS_PER_TILE)],
                    out_hbm.at[cid, pl.ds(row0, ROWS_PER_TILE)])


_INV_SQRT = float(1.0 / (1.0 + BN_EPS) ** 0.5)


def _mlp_body(x_ref, a0_ref, a1_ref, w1_ref, b1_ref, g_ref, be_ref,
              w2_ref, b2_ref, o_ref):
    z = x_ref[...] + a0_ref[0] + a1_ref[0]
    h = lax.dot_general(z, w1_ref[...], (((1,), (1,)), ((), ())),
                        preferred_element_type=jnp.float32)
    h = (h + b1_ref[...]) * (g_ref[...] * _INV_SQRT) + be_ref[...]
    h = jnp.maximum(h, 0.0)
    h = lax.dot_general(h, w2_ref[...], (((1,), (1,)), ((), ())),
                        preferred_element_type=jnp.float32)
    o_ref[...] = jnp.maximum(h + b2_ref[...], 0.0)


def _mlp(x, agg, W1, b1, gamma, beta, W2, b2):
    return pl.pallas_call(
        _mlp_body,
        grid=(GRID_R,),
        in_specs=[
            pl.BlockSpec((BR, D), lambda i: (i, 0)),
            pl.BlockSpec((1, BR, D), lambda i: (0, i, 0)),
            pl.BlockSpec((1, BR, D), lambda i: (1, i, 0)),
            pl.BlockSpec((D, D), lambda i: (0, 0)),
            pl.BlockSpec((1, D), lambda i: (0, 0)),
            pl.BlockSpec((1, D), lambda i: (0, 0)),
            pl.BlockSpec((1, D), lambda i: (0, 0)),
            pl.BlockSpec((D, D), lambda i: (0, 0)),
            pl.BlockSpec((1, D), lambda i: (0, 0)),
        ],
        out_specs=pl.BlockSpec((BR, D), lambda i: (i, 0)),
        out_shape=jax.ShapeDtypeStruct((N, D), jnp.float32),
        compiler_params=pltpu.CompilerParams(
            dimension_semantics=("arbitrary",)),
    )(x, agg, agg, W1, b1.reshape(1, D), gamma.reshape(1, D),
      beta.reshape(1, D), W2, b2.reshape(1, D))


def _head_body(h1_ref, h2_ref, h3_ref, b_ref, w1_ref, bb1_ref, w2_ref,
               bb2_ref, o_ref, acc, cnt):
    i = pl.program_id(0)

    @pl.when(i == 0)
    def _init():
        acc[...] = jnp.zeros_like(acc)
        cnt[...] = jnp.zeros_like(cnt)

    rows = lax.broadcasted_iota(jnp.int32, (BR, 1), 0) + i * BR
    valid = rows < N
    gids = lax.broadcasted_iota(jnp.int32, (BR, G), 1)
    onehot = jnp.where((b_ref[...] == gids) & valid, 1.0, 0.0)
    ones = jnp.where(jnp.broadcast_to(valid, (BR, D)), 1.0, 0.0)

    cn = (((0,), (0,)), ((), ()))
    cnt[...] += lax.dot_general(onehot, ones, cn,
                                preferred_element_type=jnp.float32)
    for k, h_ref in enumerate((h1_ref, h2_ref, h3_ref)):
        hm = jnp.where(jnp.broadcast_to(valid, (BR, D)), h_ref[...], 0.0)
        acc[:, k * D:(k + 1) * D] += lax.dot_general(
            onehot, hm, cn, preferred_element_type=jnp.float32)

    @pl.when(i == GRID_R - 1)
    def _final():
        c = jnp.maximum(cnt[...], 1.0)
        pooled = acc[...] / jnp.concatenate([c, c, c], axis=1)
        hh = lax.dot_general(pooled, w1_ref[...], (((1,), (1,)), ((), ())),
                             preferred_element_type=jnp.float32)
        hh = jnp.maximum(hh + bb1_ref[...], 0.0)
        out = lax.dot_general(hh, w2_ref[...], (((1,), (1,)), ((), ())),
                              preferred_element_type=jnp.float32)
        o_ref[...] = out + bb2_ref[...]


def _head(h1, h2, h3, batch2d, lin1_W, lin1_b, lin2_W, lin2_b):
    return pl.pallas_call(
        _head_body,
        grid=(GRID_R,),
        in_specs=[
            pl.BlockSpec((BR, D), lambda i: (i, 0)),
            pl.BlockSpec((BR, D), lambda i: (i, 0)),
            pl.BlockSpec((BR, D), lambda i: (i, 0)),
            pl.BlockSpec((BR, 1), lambda i: (i, 0)),
            pl.BlockSpec((3 * D, 3 * D), lambda i: (0, 0)),
            pl.BlockSpec((1, 3 * D), lambda i: (0, 0)),
            pl.BlockSpec((3, 3 * D), lambda i: (0, 0)),
            pl.BlockSpec((1, 3), lambda i: (0, 0)),
        ],
        out_specs=pl.BlockSpec((G, 3), lambda i: (0, 0)),
        out_shape=jax.ShapeDtypeStruct((G, 3), jnp.float32),
        scratch_shapes=[
            pltpu.VMEM((G, 3 * D), jnp.float32),
            pltpu.VMEM((G, D), jnp.float32),
        ],
        compiler_params=pltpu.CompilerParams(
            dimension_semantics=("arbitrary",)),
    )(h1, h2, h3, batch2d, lin1_W, lin1_b.reshape(1, 3 * D), lin2_W,
      lin2_b.reshape(1, 3))


def kernel(x, edge_index, batch,
           conv1_W1, conv1_b1, conv1_gamma, conv1_beta, conv1_W2, conv1_b2,
           conv2_W1, conv2_b1, conv2_gamma, conv2_beta, conv2_W2, conv2_b2,
           conv3_W1, conv3_b1, conv3_gamma, conv3_beta, conv3_W2, conv3_b2,
           lin1_W, lin1_b, lin2_W, lin2_b):
    src = edge_index[0].astype(jnp.int32)
    dst = edge_index[1].astype(jnp.int32)
    pad = E_PAD - E
    src3 = jnp.concatenate([src, jnp.zeros((pad,), jnp.int32)]).reshape(
        NW, CHUNKS_PER_TILE, CHUNK)
    dst3 = jnp.concatenate([dst, jnp.full((pad,), N, jnp.int32)]).reshape(
        NW, CHUNKS_PER_TILE, CHUNK)
    zeros = jnp.zeros((CHUNK, D), jnp.float32)
    batch2d = batch.astype(jnp.int32).reshape(N, 1)

    agg1 = _sc_scatter_add(x, src3, dst3, zeros)
    h1 = _mlp(x, agg1, conv1_W1, conv1_b1, conv1_gamma, conv1_beta,
              conv1_W2, conv1_b2)
    agg2 = _sc_scatter_add(h1, src3, dst3, zeros)
    h2 = _mlp(h1, agg2, conv2_W1, conv2_b1, conv2_gamma, conv2_beta,
              conv2_W2, conv2_b2)
    agg3 = _sc_scatter_add(h2, src3, dst3, zeros)
    h3 = _mlp(h2, agg3, conv3_W1, conv3_b1, conv3_gamma, conv3_beta,
              conv3_W2, conv3_b2)
    return _head(h1, h2, h3, batch2d, lin1_W, lin1_b, lin2_W, lin2_b)


# speedup vs baseline: 2.8389x; 1.1072x over previous
import functools

import jax
import jax.numpy as jnp
from jax import lax
from jax.experimental import pallas as pl
from jax.experimental.pallas import tpu as pltpu
from jax.experimental.pallas import tpu_sc as plsc

N = 10000
E = 320000
D = 128
G = 64
BN_EPS = 1e-5

NC, NS = 2, 16
NW = NC * NS
CHUNK = 128
CHUNKS_PER_TILE = 80
HALF_CHUNKS = CHUNKS_PER_TILE // 2
E_PAD = NW * CHUNKS_PER_TILE * CHUNK
N_PAD = 10240
ROWS_PER_TILE = N_PAD // NS
ZCOPIES = ROWS_PER_TILE // CHUNK

BR = 640
GRID_R = N_PAD // BR


_sc_mesh = plsc.VectorSubcoreMesh(
    core_axis_name="c", subcore_axis_name="s", num_cores=NC, num_subcores=NS)


@functools.partial(
    pl.kernel,
    out_type=jax.ShapeDtypeStruct((NC, N_PAD, D), jnp.float32),
    mesh=_sc_mesh,
    scratch_types=[
        pltpu.VMEM((HALF_CHUNKS, CHUNK), jnp.int32),
        pltpu.VMEM((HALF_CHUNKS, CHUNK), jnp.int32),
        pltpu.VMEM((CHUNK, D), jnp.float32),
        pltpu.VMEM((CHUNK, D), jnp.float32),
        pltpu.VMEM_SHARED((N_PAD, D), jnp.float32),
        pltpu.SemaphoreType.DMA,
        pltpu.SemaphoreType.DMA,
    ],
)
def _sc_scatter_add(x_hbm, src_hbm, dst_hbm, zero_hbm, out_hbm,
                    src_v, dst_v, rows0, rows1, acc_sh, sem0, sem1):
    cid = lax.axis_index("c")
    sid = lax.axis_index("s")
    wid = cid * NS + sid
    row0 = sid * ROWS_PER_TILE

    pltpu.sync_copy(zero_hbm.at[pl.ds(0, CHUNK)], rows0)
    for k in range(ZCOPIES):
        pltpu.sync_copy(rows0, acc_sh.at[pl.ds(row0 + k * CHUNK, CHUNK)])
    plsc.subcore_barrier()

    for half in range(2):
        pltpu.sync_copy(src_hbm.at[wid, pl.ds(half * HALF_CHUNKS, HALF_CHUNKS)],
                        src_v)
        pltpu.sync_copy(dst_hbm.at[wid, pl.ds(half * HALF_CHUNKS, HALF_CHUNKS)],
                        dst_v)
        pltpu.async_copy(x_hbm.at[src_v.at[0]], rows0, sem0)
        pltpu.async_copy(x_hbm.at[src_v.at[1]], rows1, sem1)

        def body(p, carry):
            j0 = 2 * p
            pltpu.make_async_copy(x_hbm.at[src_v.at[0]], rows0, sem0).wait()
            pltpu.sync_copy(rows0, acc_sh.at[dst_v.at[j0]], add=True)
            pltpu.async_copy(x_hbm.at[src_v.at[j0 + 2]], rows0, sem0)
            pltpu.make_async_copy(x_hbm.at[src_v.at[1]], rows1, sem1).wait()
            pltpu.sync_copy(rows1, acc_sh.at[dst_v.at[j0 + 1]], add=True)
            pltpu.async_copy(x_hbm.at[src_v.at[j0 + 3]], rows1, sem1)
            return carry

        lax.fori_loop(0, HALF_CHUNKS // 2 - 1, body, 0)
        pltpu.make_async_copy(x_hbm.at[src_v.at[0]], rows0, sem0).wait()
        pltpu.sync_copy(rows0, acc_sh.at[dst_v.at[HALF_CHUNKS - 2]], add=True)
        pltpu.make_async_copy(x_hbm.at[src_v.at[1]], rows1, sem1).wait()
        pltpu.sync_copy(rows1, acc_sh.at[dst_v.at[HALF_CHUNKS - 1]], add=True)
    plsc.subcore_barrier()

    pltpu.sync_copy(acc_sh.at[pl.ds(row0, ROWS_PER_TILE)],
                    out_hbm.at[cid, pl.ds(row0, ROWS_PER_TILE)])


_INV_SQRT = float(1.0 / (1.0 + BN_EPS) ** 0.5)


def _mlp_body(x_ref, a0_ref, a1_ref, w1_ref, b1_ref, g_ref, be_ref,
              w2_ref, b2_ref, o_ref):
    z = x_ref[...] + a0_ref[0] + a1_ref[0]
    h = lax.dot_general(z, w1_ref[...], (((1,), (1,)), ((), ())),
                        preferred_element_type=jnp.float32)
    h = (h + b1_ref[...]) * (g_ref[...] * _INV_SQRT) + be_ref[...]
    h = jnp.maximum(h, 0.0)
    h = lax.dot_general(h, w2_ref[...], (((1,), (1,)), ((), ())),
                        preferred_element_type=jnp.float32)
    o_ref[...] = jnp.maximum(h + b2_ref[...], 0.0)


def _mlp(x, agg, W1, b1, gamma, beta, W2, b2):
    return pl.pallas_call(
        _mlp_body,
        grid=(GRID_R,),
        in_specs=[
            pl.BlockSpec((BR, D), lambda i: (i, 0)),
            pl.BlockSpec((1, BR, D), lambda i: (0, i, 0)),
            pl.BlockSpec((1, BR, D), lambda i: (1, i, 0)),
            pl.BlockSpec((D, D), lambda i: (0, 0)),
            pl.BlockSpec((1, D), lambda i: (0, 0)),
            pl.BlockSpec((1, D), lambda i: (0, 0)),
            pl.BlockSpec((1, D), lambda i: (0, 0)),
            pl.BlockSpec((D, D), lambda i: (0, 0)),
            pl.BlockSpec((1, D), lambda i: (0, 0)),
        ],
        out_specs=pl.BlockSpec((BR, D), lambda i: (i, 0)),
        out_shape=jax.ShapeDtypeStruct((N, D), jnp.float32),
        compiler_params=pltpu.CompilerParams(
            dimension_semantics=("arbitrary",)),
    )(x, agg, agg, W1, b1.reshape(1, D), gamma.reshape(1, D),
      beta.reshape(1, D), W2, b2.reshape(1, D))


def _head_body(h1_ref, h2_ref, h3_ref, b_ref, w1_ref, bb1_ref, w2_ref,
               bb2_ref, o_ref, acc, cnt):
    i = pl.program_id(0)

    @pl.when(i == 0)
    def _init():
        acc[...] = jnp.zeros_like(acc)
        cnt[...] = jnp.zeros_like(cnt)

    rows = lax.broadcasted_iota(jnp.int32, (BR, 1), 0) + i * BR
    valid = rows < N
    gids = lax.broadcasted_iota(jnp.int32, (BR, G), 1)
    onehot = jnp.where((b_ref[...] == gids) & valid, 1.0, 0.0)
    ones = jnp.where(jnp.broadcast_to(valid, (BR, D)), 1.0, 0.0)

    cn = (((0,), (0,)), ((), ()))
    cnt[...] += lax.dot_general(onehot, ones, cn,
                                preferred_element_type=jnp.float32)
    for k, h_ref in enumerate((h1_ref, h2_ref, h3_ref)):
        hm = jnp.where(jnp.broadcast_to(valid, (BR, D)), h_ref[...], 0.0)
        acc[:, k * D:(k + 1) * D] += lax.dot_general(
            onehot, hm, cn, preferred_element_type=jnp.float32)

    @pl.when(i == GRID_R - 1)
    def _final():
        c = jnp.maximum(cnt[...], 1.0)
        pooled = acc[...] / jnp.concatenate([c, c, c], axis=1)
        hh = lax.dot_general(pooled, w1_ref[...], (((1,), (1,)), ((), ())),
                             preferred_element_type=jnp.float32)
        hh = jnp.maximum(hh + bb1_ref[...], 0.0)
        out = lax.dot_general(hh, w2_ref[...], (((1,), (1,)), ((), ())),
                              preferred_element_type=jnp.float32)
        o_ref[...] = out + bb2_ref[...]


def _head(h1, h2, h3, batch2d, lin1_W, lin1_b, lin2_W, lin2_b):
    return pl.pallas_call(
        _head_body,
        grid=(GRID_R,),
        in_specs=[
            pl.BlockSpec((BR, D), lambda i: (i, 0)),
            pl.BlockSpec((BR, D), lambda i: (i, 0)),
            pl.BlockSpec((BR, D), lambda i: (i, 0)),
            pl.BlockSpec((BR, 1), lambda i: (i, 0)),
            pl.BlockSpec((3 * D, 3 * D), lambda i: (0, 0)),
            pl.BlockSpec((1, 3 * D), lambda i: (0, 0)),
            pl.BlockSpec((3, 3 * D), lambda i: (0, 0)),
            pl.BlockSpec((1, 3), lambda i: (0, 0)),
        ],
        out_specs=pl.BlockSpec((G, 3), lambda i: (0, 0)),
        out_shape=jax.ShapeDtypeStruct((G, 3), jnp.float32),
        scratch_shapes=[
            pltpu.VMEM((G, 3 * D), jnp.float32),
            pltpu.VMEM((G, D), jnp.float32),
        ],
        compiler_params=pltpu.CompilerParams(
            dimension_semantics=("arbitrary",)),
    )(h1, h2, h3, batch2d, lin1_W, lin1_b.reshape(1, 3 * D), lin2_W,
      lin2_b.reshape(1, 3))


def kernel(x, edge_index, batch,
           conv1_W1, conv1_b1, conv1_gamma, conv1_beta, conv1_W2, conv1_b2,
           conv2_W1, conv2_b1, conv2_gamma, conv2_beta, conv2_W2, conv2_b2,
           conv3_W1, conv3_b1, conv3_gamma, conv3_beta, conv3_W2, conv3_b2,
           lin1_W, lin1_b, lin2_W, lin2_b):
    src = edge_index[0].astype(jnp.int32)
    dst = edge_index[1].astype(jnp.int32)
    pad = E_PAD - E
    src3 = jnp.concatenate([src, jnp.zeros((pad,), jnp.int32)]).reshape(
        NW, CHUNKS_PER_TILE, CHUNK)
    dst3 = jnp.concatenate([dst, jnp.full((pad,), N, jnp.int32)]).reshape(
        NW, CHUNKS_PER_TILE, CHUNK)
    zeros = jnp.zeros((CHUNK, D), jnp.float32)
    batch2d = batch.astype(jnp.int32).reshape(N, 1)

    agg1 = _sc_scatter_add(x, src3, dst3, zeros)
    h1 = _mlp(x, agg1, conv1_W1, conv1_b1, conv1_gamma, conv1_beta,
              conv1_W2, conv1_b2)
    agg2 = _sc_scatter_add(h1, src3, dst3, zeros)
    h2 = _mlp(h1, agg2, conv2_W1, conv2_b1, conv2_gamma, conv2_beta,
              conv2_W2, conv2_b2)
    agg3 = _sc_scatter_add(h2, src3, dst3, zeros)
    h3 = _mlp(h2, agg3, conv3_W1, conv3_b1, conv3_gamma, conv3_beta,
              conv3_W2, conv3_b2)
    return _head(h1, h2, h3, batch2d, lin1_W, lin1_b, lin2_W, lin2_b)


# speedup vs baseline: 2.8420x; 1.0011x over previous
import functools

import jax
import jax.numpy as jnp
from jax import lax
from jax.experimental import pallas as pl
from jax.experimental.pallas import tpu as pltpu
from jax.experimental.pallas import tpu_sc as plsc

N = 10000
E = 320000
D = 128
G = 64
BN_EPS = 1e-5

NC, NS = 2, 16
NW = NC * NS
CHUNK = 128
CHUNKS_PER_TILE = 80
HALF_CHUNKS = CHUNKS_PER_TILE // 2
E_PAD = NW * CHUNKS_PER_TILE * CHUNK
N_PAD = 10240
ROWS_PER_TILE = N_PAD // NS
ZCOPIES = ROWS_PER_TILE // CHUNK

BR = 640
GRID_R = N_PAD // BR


_sc_mesh = plsc.VectorSubcoreMesh(
    core_axis_name="c", subcore_axis_name="s", num_cores=NC, num_subcores=NS)


@functools.partial(
    pl.kernel,
    out_type=jax.ShapeDtypeStruct((NC, N_PAD, D), jnp.float32),
    mesh=_sc_mesh,
    scratch_types=[
        pltpu.VMEM((HALF_CHUNKS, CHUNK), jnp.int32),
        pltpu.VMEM((HALF_CHUNKS, CHUNK), jnp.int32),
        pltpu.VMEM((CHUNK, D), jnp.float32),
        pltpu.VMEM((CHUNK, D), jnp.float32),
        pltpu.VMEM_SHARED((N_PAD, D), jnp.float32),
        pltpu.SemaphoreType.DMA,
        pltpu.SemaphoreType.DMA,
    ],
)
def _sc_scatter_add(x_hbm, src_hbm, dst_hbm, zero_hbm, out_hbm,
                    src_v, dst_v, rows0, rows1, acc_sh, sem0, sem1):
    cid = lax.axis_index("c")
    sid = lax.axis_index("s")
    wid = cid * NS + sid
    row0 = sid * ROWS_PER_TILE

    pltpu.sync_copy(zero_hbm.at[pl.ds(0, CHUNK)], rows0)
    for k in range(ZCOPIES):
        pltpu.sync_copy(rows0, acc_sh.at[pl.ds(row0 + k * CHUNK, CHUNK)])
    plsc.subcore_barrier()

    for half in range(2):
        pltpu.sync_copy(src_hbm.at[wid, pl.ds(half * HALF_CHUNKS, HALF_CHUNKS)],
                        src_v)
        pltpu.sync_copy(dst_hbm.at[wid, pl.ds(half * HALF_CHUNKS, HALF_CHUNKS)],
                        dst_v)
        pltpu.async_copy(x_hbm.at[src_v.at[0]], rows0, sem0)
        pltpu.async_copy(x_hbm.at[src_v.at[1]], rows1, sem1)

        def body(p, carry):
            j0 = 2 * p
            pltpu.make_async_copy(x_hbm.at[src_v.at[0]], rows0, sem0).wait()
            pltpu.sync_copy(rows0, acc_sh.at[dst_v.at[j0]], add=True)
            pltpu.async_copy(x_hbm.at[src_v.at[j0 + 2]], rows0, sem0)
            pltpu.make_async_copy(x_hbm.at[src_v.at[1]], rows1, sem1).wait()
            pltpu.sync_copy(rows1, acc_sh.at[dst_v.at[j0 + 1]], add=True)
            pltpu.async_copy(x_hbm.at[src_v.at[j0 + 3]], rows1, sem1)
            return carry

        lax.fori_loop(0, HALF_CHUNKS // 2 - 1, body, 0)
        pltpu.make_async_copy(x_hbm.at[src_v.at[0]], rows0, sem0).wait()
        pltpu.sync_copy(rows0, acc_sh.at[dst_v.at[HALF_CHUNKS - 2]], add=True)
        pltpu.make_async_copy(x_hbm.at[src_v.at[1]], rows1, sem1).wait()
        pltpu.sync_copy(rows1, acc_sh.at[dst_v.at[HALF_CHUNKS - 1]], add=True)
    plsc.subcore_barrier()

    pltpu.sync_copy(acc_sh.at[pl.ds(row0, ROWS_PER_TILE)],
                    out_hbm.at[cid, pl.ds(row0, ROWS_PER_TILE)])


_INV_SQRT = float(1.0 / (1.0 + BN_EPS) ** 0.5)


def _mlp_body(x_ref, a0_ref, a1_ref, w1_ref, b1_ref, g_ref, be_ref,
              w2_ref, b2_ref, o_ref):
    z = x_ref[...] + a0_ref[0] + a1_ref[0]
    h = lax.dot_general(z, w1_ref[...], (((1,), (1,)), ((), ())),
                        preferred_element_type=jnp.float32)
    h = (h + b1_ref[...]) * (g_ref[...] * _INV_SQRT) + be_ref[...]
    h = jnp.maximum(h, 0.0)
    h = lax.dot_general(h, w2_ref[...], (((1,), (1,)), ((), ())),
                        preferred_element_type=jnp.float32)
    o_ref[...] = jnp.maximum(h + b2_ref[...], 0.0)


def _mlp(x, agg, W1, b1, gamma, beta, W2, b2):
    return pl.pallas_call(
        _mlp_body,
        grid=(GRID_R,),
        in_specs=[
            pl.BlockSpec((BR, D), lambda i: (i, 0)),
            pl.BlockSpec((1, BR, D), lambda i: (0, i, 0)),
            pl.BlockSpec((1, BR, D), lambda i: (1, i, 0)),
            pl.BlockSpec((D, D), lambda i: (0, 0)),
            pl.BlockSpec((1, D), lambda i: (0, 0)),
            pl.BlockSpec((1, D), lambda i: (0, 0)),
            pl.BlockSpec((1, D), lambda i: (0, 0)),
            pl.BlockSpec((D, D), lambda i: (0, 0)),
            pl.BlockSpec((1, D), lambda i: (0, 0)),
        ],
        out_specs=pl.BlockSpec((BR, D), lambda i: (i, 0)),
        out_shape=jax.ShapeDtypeStruct((N, D), jnp.float32),
        compiler_params=pltpu.CompilerParams(
            dimension_semantics=("arbitrary",)),
    )(x, agg, agg, W1, b1.reshape(1, D), gamma.reshape(1, D),
      beta.reshape(1, D), W2, b2.reshape(1, D))


def _head_body(h1_ref, h2_ref, h3_ref, b_ref, w1_ref, bb1_ref, w2_ref,
               bb2_ref, o_ref, acc, cnt):
    i = pl.program_id(0)

    @pl.when(i == 0)
    def _init():
        acc[...] = jnp.zeros_like(acc)
        cnt[...] = jnp.zeros_like(cnt)

    rows = lax.broadcasted_iota(jnp.int32, (BR, 1), 0) + i * BR
    valid = rows < N
    gids = lax.broadcasted_iota(jnp.int32, (BR, G), 1)
    onehot = jnp.where((b_ref[...] == gids) & valid, 1.0, 0.0)
    ones = jnp.where(jnp.broadcast_to(valid, (BR, D)), 1.0, 0.0)

    cn = (((0,), (0,)), ((), ()))
    cnt[...] += lax.dot_general(onehot, ones, cn,
                                preferred_element_type=jnp.float32)
    for k, h_ref in enumerate((h1_ref, h2_ref, h3_ref)):
        hm = jnp.where(jnp.broadcast_to(valid, (BR, D)), h_ref[...], 0.0)
        acc[:, k * D:(k + 1) * D] += lax.dot_general(
            onehot, hm, cn, preferred_element_type=jnp.float32)

    @pl.when(i == GRID_R - 1)
    def _final():
        c = jnp.maximum(cnt[...], 1.0)
        pooled = acc[...] / jnp.concatenate([c, c, c], axis=1)
        hh = lax.dot_general(pooled, w1_ref[...], (((1,), (1,)), ((), ())),
                             preferred_element_type=jnp.float32)
        hh = jnp.maximum(hh + bb1_ref[...], 0.0)
        out = lax.dot_general(hh, w2_ref[...], (((1,), (1,)), ((), ())),
                              preferred_element_type=jnp.float32)
        o_ref[...] = out + bb2_ref[...]


def _head(h1, h2, h3, batch2d, lin1_W, lin1_b, lin2_W, lin2_b):
    return pl.pallas_call(
        _head_body,
        grid=(GRID_R,),
        in_specs=[
            pl.BlockSpec((BR, D), lambda i: (i, 0)),
            pl.BlockSpec((BR, D), lambda i: (i, 0)),
            pl.BlockSpec((BR, D), lambda i: (i, 0)),
            pl.BlockSpec((BR, 1), lambda i: (i, 0)),
            pl.BlockSpec((3 * D, 3 * D), lambda i: (0, 0)),
            pl.BlockSpec((1, 3 * D), lambda i: (0, 0)),
            pl.BlockSpec((3, 3 * D), lambda i: (0, 0)),
            pl.BlockSpec((1, 3), lambda i: (0, 0)),
        ],
        out_specs=pl.BlockSpec((G, 3), lambda i: (0, 0)),
        out_shape=jax.ShapeDtypeStruct((G, 3), jnp.float32),
        scratch_shapes=[
            pltpu.VMEM((G, 3 * D), jnp.float32),
            pltpu.VMEM((G, D), jnp.float32),
        ],
        compiler_params=pltpu.CompilerParams(
            dimension_semantics=("arbitrary",)),
    )(h1, h2, h3, batch2d, lin1_W, lin1_b.reshape(1, 3 * D), lin2_W,
      lin2_b.reshape(1, 3))


def kernel(x, edge_index, batch,
           conv1_W1, conv1_b1, conv1_gamma, conv1_beta, conv1_W2, conv1_b2,
           conv2_W1, conv2_b1, conv2_gamma, conv2_beta, conv2_W2, conv2_b2,
           conv3_W1, conv3_b1, conv3_gamma, conv3_beta, conv3_W2, conv3_b2,
           lin1_W, lin1_b, lin2_W, lin2_b):
    src = edge_index[0].astype(jnp.int32)
    dst = edge_index[1].astype(jnp.int32)
    pad = E_PAD - E
    src3 = jnp.concatenate([src, jnp.zeros((pad,), jnp.int32)]).reshape(
        NW, CHUNKS_PER_TILE, CHUNK)
    trash = N + (jnp.arange(pad, dtype=jnp.int32) % (N_PAD - N))
    dst3 = jnp.concatenate([dst, trash]).reshape(
        NW, CHUNKS_PER_TILE, CHUNK)
    zeros = jnp.zeros((CHUNK, D), jnp.float32)
    batch2d = batch.astype(jnp.int32).reshape(N, 1)

    agg1 = _sc_scatter_add(x, src3, dst3, zeros)
    h1 = _mlp(x, agg1, conv1_W1, conv1_b1, conv1_gamma, conv1_beta,
              conv1_W2, conv1_b2)
    agg2 = _sc_scatter_add(h1, src3, dst3, zeros)
    h2 = _mlp(h1, agg2, conv2_W1, conv2_b1, conv2_gamma, conv2_beta,
              conv2_W2, conv2_b2)
    agg3 = _sc_scatter_add(h2, src3, dst3, zeros)
    h3 = _mlp(h2, agg3, conv3_W1, conv3_b1, conv3_gamma, conv3_beta,
              conv3_W2, conv3_b2)
    return _head(h1, h2, h3, batch2d, lin1_W, lin1_b, lin2_W, lin2_b)


# speedup vs baseline: 10.1032x; 3.5549x over previous
import functools

import jax
import jax.numpy as jnp
from jax import lax
from jax.experimental import pallas as pl
from jax.experimental.pallas import tpu as pltpu
from jax.experimental.pallas import tpu_sc as plsc

N = 10000
E = 320000
D = 128
G = 64
BN_EPS = 1e-5

NC, NS = 2, 16
NW = NC * NS
CHUNK = 128
CHUNKS_PER_TILE = 80
HALF_CHUNKS = CHUNKS_PER_TILE // 2
E_PAD = NW * CHUNKS_PER_TILE * CHUNK
N_PAD = 10240
ROWS_PER_TILE = N_PAD // NS
ZCOPIES = ROWS_PER_TILE // CHUNK

BR = 640
GRID_R = N_PAD // BR


_sc_mesh = plsc.VectorSubcoreMesh(
    core_axis_name="c", subcore_axis_name="s", num_cores=NC, num_subcores=NS)


@functools.partial(
    pl.kernel,
    out_type=jax.ShapeDtypeStruct((NC, N_PAD, D), jnp.float32),
    mesh=_sc_mesh,
    scratch_types=[
        pltpu.VMEM((HALF_CHUNKS, CHUNK), jnp.int32),
        pltpu.VMEM((HALF_CHUNKS, CHUNK), jnp.int32),
        pltpu.VMEM((CHUNK, D), jnp.float32),
        pltpu.VMEM((CHUNK, D), jnp.float32),
        pltpu.VMEM_SHARED((N_PAD, D), jnp.float32),
        pltpu.SemaphoreType.DMA,
        pltpu.SemaphoreType.DMA,
    ],
)
def _sc_scatter_add(x_hbm, src_hbm, dst_hbm, zero_hbm, out_hbm,
                    src_v, dst_v, rows0, rows1, acc_sh, sem0, sem1):
    cid = lax.axis_index("c")
    sid = lax.axis_index("s")
    wid = cid * NS + sid
    row0 = sid * ROWS_PER_TILE

    pltpu.sync_copy(zero_hbm.at[pl.ds(0, CHUNK)], rows0)
    for k in range(ZCOPIES):
        pltpu.sync_copy(rows0, acc_sh.at[pl.ds(row0 + k * CHUNK, CHUNK)])
    plsc.subcore_barrier()

    for half in range(2):
        pltpu.sync_copy(src_hbm.at[wid, pl.ds(half * HALF_CHUNKS, HALF_CHUNKS)],
                        src_v)
        pltpu.sync_copy(dst_hbm.at[wid, pl.ds(half * HALF_CHUNKS, HALF_CHUNKS)],
                        dst_v)
        pltpu.async_copy(x_hbm.at[src_v.at[0]], rows0, sem0)
        pltpu.async_copy(x_hbm.at[src_v.at[1]], rows1, sem1)

        def body(p, carry):
            j0 = 2 * p
            pltpu.make_async_copy(x_hbm.at[src_v.at[0]], rows0, sem0).wait()
            pltpu.sync_copy(rows0, acc_sh.at[dst_v.at[j0]], add=True)
            pltpu.async_copy(x_hbm.at[src_v.at[j0 + 2]], rows0, sem0)
            pltpu.make_async_copy(x_hbm.at[src_v.at[1]], rows1, sem1).wait()
            pltpu.sync_copy(rows1, acc_sh.at[dst_v.at[j0 + 1]], add=True)
            pltpu.async_copy(x_hbm.at[src_v.at[j0 + 3]], rows1, sem1)
            return carry

        lax.fori_loop(0, HALF_CHUNKS // 2 - 1, body, 0)
        pltpu.make_async_copy(x_hbm.at[src_v.at[0]], rows0, sem0).wait()
        pltpu.sync_copy(rows0, acc_sh.at[dst_v.at[HALF_CHUNKS - 2]], add=True)
        pltpu.make_async_copy(x_hbm.at[src_v.at[1]], rows1, sem1).wait()
        pltpu.sync_copy(rows1, acc_sh.at[dst_v.at[HALF_CHUNKS - 1]], add=True)
    plsc.subcore_barrier()

    pltpu.sync_copy(acc_sh.at[pl.ds(row0, ROWS_PER_TILE)],
                    out_hbm.at[cid, pl.ds(row0, ROWS_PER_TILE)])


_INV_SQRT = float(1.0 / (1.0 + BN_EPS) ** 0.5)


def _mlp_body(x_ref, a0_ref, a1_ref, w1_ref, b1_ref, g_ref, be_ref,
              w2_ref, b2_ref, o_ref):
    z = x_ref[...] + a0_ref[0] + a1_ref[0]
    h = lax.dot_general(z, w1_ref[...], (((1,), (1,)), ((), ())),
                        preferred_element_type=jnp.float32)
    h = (h + b1_ref[...]) * (g_ref[...] * _INV_SQRT) + be_ref[...]
    h = jnp.maximum(h, 0.0)
    h = lax.dot_general(h, w2_ref[...], (((1,), (1,)), ((), ())),
                        preferred_element_type=jnp.float32)
    o_ref[...] = jnp.maximum(h + b2_ref[...], 0.0)


def _mlp(x, agg, W1, b1, gamma, beta, W2, b2):
    return pl.pallas_call(
        _mlp_body,
        grid=(GRID_R,),
        in_specs=[
            pl.BlockSpec((BR, D), lambda i: (i, 0)),
            pl.BlockSpec((1, BR, D), lambda i: (0, i, 0)),
            pl.BlockSpec((1, BR, D), lambda i: (1, i, 0)),
            pl.BlockSpec((D, D), lambda i: (0, 0)),
            pl.BlockSpec((1, D), lambda i: (0, 0)),
            pl.BlockSpec((1, D), lambda i: (0, 0)),
            pl.BlockSpec((1, D), lambda i: (0, 0)),
            pl.BlockSpec((D, D), lambda i: (0, 0)),
            pl.BlockSpec((1, D), lambda i: (0, 0)),
        ],
        out_specs=pl.BlockSpec((BR, D), lambda i: (i, 0)),
        out_shape=jax.ShapeDtypeStruct((N, D), jnp.float32),
        compiler_params=pltpu.CompilerParams(
            dimension_semantics=("arbitrary",)),
    )(x, agg, agg, W1, b1.reshape(1, D), gamma.reshape(1, D),
      beta.reshape(1, D), W2, b2.reshape(1, D))


def _head_body(h1_ref, h2_ref, h3_ref, b_ref, w1_ref, bb1_ref, w2_ref,
               bb2_ref, o_ref, acc, cnt):
    i = pl.program_id(0)

    @pl.when(i == 0)
    def _init():
        acc[...] = jnp.zeros_like(acc)
        cnt[...] = jnp.zeros_like(cnt)

    rows = lax.broadcasted_iota(jnp.int32, (BR, 1), 0) + i * BR
    valid = rows < N
    gids = lax.broadcasted_iota(jnp.int32, (BR, G), 1)
    onehot = jnp.where((b_ref[...] == gids) & valid, 1.0, 0.0)
    ones = jnp.where(jnp.broadcast_to(valid, (BR, D)), 1.0, 0.0)

    cn = (((0,), (0,)), ((), ()))
    cnt[...] += lax.dot_general(onehot, ones, cn,
                                preferred_element_type=jnp.float32)
    for k, h_ref in enumerate((h1_ref, h2_ref, h3_ref)):
        hm = jnp.where(jnp.broadcast_to(valid, (BR, D)), h_ref[...], 0.0)
        acc[:, k * D:(k + 1) * D] += lax.dot_general(
            onehot, hm, cn, preferred_element_type=jnp.float32)

    @pl.when(i == GRID_R - 1)
    def _final():
        c = jnp.maximum(cnt[...], 1.0)
        pooled = acc[...] / jnp.concatenate([c, c, c], axis=1)
        hh = lax.dot_general(pooled, w1_ref[...], (((1,), (1,)), ((), ())),
                             preferred_element_type=jnp.float32)
        hh = jnp.maximum(hh + bb1_ref[...], 0.0)
        out = lax.dot_general(hh, w2_ref[...], (((1,), (1,)), ((), ())),
                              preferred_element_type=jnp.float32)
        o_ref[...] = out + bb2_ref[...]


def _head(h1, h2, h3, batch2d, lin1_W, lin1_b, lin2_W, lin2_b):
    return pl.pallas_call(
        _head_body,
        grid=(GRID_R,),
        in_specs=[
            pl.BlockSpec((BR, D), lambda i: (i, 0)),
            pl.BlockSpec((BR, D), lambda i: (i, 0)),
            pl.BlockSpec((BR, D), lambda i: (i, 0)),
            pl.BlockSpec((BR, 1), lambda i: (i, 0)),
            pl.BlockSpec((3 * D, 3 * D), lambda i: (0, 0)),
            pl.BlockSpec((1, 3 * D), lambda i: (0, 0)),
            pl.BlockSpec((3, 3 * D), lambda i: (0, 0)),
            pl.BlockSpec((1, 3), lambda i: (0, 0)),
        ],
        out_specs=pl.BlockSpec((G, 3), lambda i: (0, 0)),
        out_shape=jax.ShapeDtypeStruct((G, 3), jnp.float32),
        scratch_shapes=[
            pltpu.VMEM((G, 3 * D), jnp.float32),
            pltpu.VMEM((G, D), jnp.float32),
        ],
        compiler_params=pltpu.CompilerParams(
            dimension_semantics=("arbitrary",)),
    )(h1, h2, h3, batch2d, lin1_W, lin1_b.reshape(1, 3 * D), lin2_W,
      lin2_b.reshape(1, 3))


def kernel(x, edge_index, batch,
           conv1_W1, conv1_b1, conv1_gamma, conv1_beta, conv1_W2, conv1_b2,
           conv2_W1, conv2_b1, conv2_gamma, conv2_beta, conv2_W2, conv2_b2,
           conv3_W1, conv3_b1, conv3_gamma, conv3_beta, conv3_W2, conv3_b2,
           lin1_W, lin1_b, lin2_W, lin2_b):
    src = edge_index[0].astype(jnp.int32)
    dst = edge_index[1].astype(jnp.int32)
    pad = E_PAD - E
    padsrc = jnp.arange(pad, dtype=jnp.int32) % N
    src3 = jnp.concatenate([src, padsrc]).reshape(
        NW, CHUNKS_PER_TILE, CHUNK)
    trash = N + (jnp.arange(pad, dtype=jnp.int32) % (N_PAD - N))
    dst3 = jnp.concatenate([dst, trash]).reshape(
        NW, CHUNKS_PER_TILE, CHUNK)
    zeros = jnp.zeros((CHUNK, D), jnp.float32)
    batch2d = batch.astype(jnp.int32).reshape(N, 1)

    agg1 = _sc_scatter_add(x, src3, dst3, zeros)
    h1 = _mlp(x, agg1, conv1_W1, conv1_b1, conv1_gamma, conv1_beta,
              conv1_W2, conv1_b2)
    agg2 = _sc_scatter_add(h1, src3, dst3, zeros)
    h2 = _mlp(h1, agg2, conv2_W1, conv2_b1, conv2_gamma, conv2_beta,
              conv2_W2, conv2_b2)
    agg3 = _sc_scatter_add(h2, src3, dst3, zeros)
    h3 = _mlp(h2, agg3, conv3_W1, conv3_b1, conv3_gamma, conv3_beta,
              conv3_W2, conv3_b2)
    return _head(h1, h2, h3, batch2d, lin1_W, lin1_b, lin2_W, lin2_b)


# speedup vs baseline: 10.7293x; 1.0620x over previous
import functools

import jax
import jax.numpy as jnp
from jax import lax
from jax.experimental import pallas as pl
from jax.experimental.pallas import tpu as pltpu
from jax.experimental.pallas import tpu_sc as plsc

N = 10000
E = 320000
D = 128
G = 64
BN_EPS = 1e-5

NC, NS = 2, 16
NW = NC * NS
CHUNK = 128
CHUNKS_PER_TILE = 80
HALF_CHUNKS = CHUNKS_PER_TILE // 2
E_PAD = NW * CHUNKS_PER_TILE * CHUNK
N_PAD = 10240
ROWS_PER_TILE = N_PAD // NS
ZCOPIES = ROWS_PER_TILE // CHUNK

BR = 2048
GRID_R = N_PAD // BR


_sc_mesh = plsc.VectorSubcoreMesh(
    core_axis_name="c", subcore_axis_name="s", num_cores=NC, num_subcores=NS)


@functools.partial(
    pl.kernel,
    out_type=jax.ShapeDtypeStruct((NC, N_PAD, D), jnp.float32),
    mesh=_sc_mesh,
    scratch_types=[
        pltpu.VMEM((HALF_CHUNKS, CHUNK), jnp.int32),
        pltpu.VMEM((HALF_CHUNKS, CHUNK), jnp.int32),
        pltpu.VMEM((CHUNK, D), jnp.float32),
        pltpu.VMEM((CHUNK, D), jnp.float32),
        pltpu.VMEM_SHARED((N_PAD, D), jnp.float32),
        pltpu.SemaphoreType.DMA,
        pltpu.SemaphoreType.DMA,
    ],
)
def _sc_scatter_add(x_hbm, src_hbm, dst_hbm, zero_hbm, out_hbm,
                    src_v, dst_v, rows0, rows1, acc_sh, sem0, sem1):
    cid = lax.axis_index("c")
    sid = lax.axis_index("s")
    wid = cid * NS + sid
    row0 = sid * ROWS_PER_TILE

    pltpu.sync_copy(zero_hbm.at[pl.ds(0, CHUNK)], rows0)
    for k in range(ZCOPIES):
        pltpu.sync_copy(rows0, acc_sh.at[pl.ds(row0 + k * CHUNK, CHUNK)])
    plsc.subcore_barrier()

    for half in range(2):
        pltpu.sync_copy(src_hbm.at[wid, pl.ds(half * HALF_CHUNKS, HALF_CHUNKS)],
                        src_v)
        pltpu.sync_copy(dst_hbm.at[wid, pl.ds(half * HALF_CHUNKS, HALF_CHUNKS)],
                        dst_v)
        pltpu.async_copy(x_hbm.at[src_v.at[0]], rows0, sem0)
        pltpu.async_copy(x_hbm.at[src_v.at[1]], rows1, sem1)

        def body(p, carry):
            j0 = 2 * p
            pltpu.make_async_copy(x_hbm.at[src_v.at[0]], rows0, sem0).wait()
            pltpu.sync_copy(rows0, acc_sh.at[dst_v.at[j0]], add=True)
            pltpu.async_copy(x_hbm.at[src_v.at[j0 + 2]], rows0, sem0)
            pltpu.make_async_copy(x_hbm.at[src_v.at[1]], rows1, sem1).wait()
            pltpu.sync_copy(rows1, acc_sh.at[dst_v.at[j0 + 1]], add=True)
            pltpu.async_copy(x_hbm.at[src_v.at[j0 + 3]], rows1, sem1)
            return carry

        lax.fori_loop(0, HALF_CHUNKS // 2 - 1, body, 0)
        pltpu.make_async_copy(x_hbm.at[src_v.at[0]], rows0, sem0).wait()
        pltpu.sync_copy(rows0, acc_sh.at[dst_v.at[HALF_CHUNKS - 2]], add=True)
        pltpu.make_async_copy(x_hbm.at[src_v.at[1]], rows1, sem1).wait()
        pltpu.sync_copy(rows1, acc_sh.at[dst_v.at[HALF_CHUNKS - 1]], add=True)
    plsc.subcore_barrier()

    pltpu.sync_copy(acc_sh.at[pl.ds(row0, ROWS_PER_TILE)],
                    out_hbm.at[cid, pl.ds(row0, ROWS_PER_TILE)])


_INV_SQRT = float(1.0 / (1.0 + BN_EPS) ** 0.5)


def _mlp_body(x_ref, a0_ref, a1_ref, w1_ref, b1_ref, g_ref, be_ref,
              w2_ref, b2_ref, o_ref):
    z = x_ref[...] + a0_ref[0] + a1_ref[0]
    h = lax.dot_general(z, w1_ref[...], (((1,), (1,)), ((), ())),
                        preferred_element_type=jnp.float32)
    h = (h + b1_ref[...]) * (g_ref[...] * _INV_SQRT) + be_ref[...]
    h = jnp.maximum(h, 0.0)
    h = lax.dot_general(h, w2_ref[...], (((1,), (1,)), ((), ())),
                        preferred_element_type=jnp.float32)
    o_ref[...] = jnp.maximum(h + b2_ref[...], 0.0)


def _mlp(x, agg, W1, b1, gamma, beta, W2, b2):
    return pl.pallas_call(
        _mlp_body,
        grid=(GRID_R,),
        in_specs=[
            pl.BlockSpec((BR, D), lambda i: (i, 0)),
            pl.BlockSpec((1, BR, D), lambda i: (0, i, 0)),
            pl.BlockSpec((1, BR, D), lambda i: (1, i, 0)),
            pl.BlockSpec((D, D), lambda i: (0, 0)),
            pl.BlockSpec((1, D), lambda i: (0, 0)),
            pl.BlockSpec((1, D), lambda i: (0, 0)),
            pl.BlockSpec((1, D), lambda i: (0, 0)),
            pl.BlockSpec((D, D), lambda i: (0, 0)),
            pl.BlockSpec((1, D), lambda i: (0, 0)),
        ],
        out_specs=pl.BlockSpec((BR, D), lambda i: (i, 0)),
        out_shape=jax.ShapeDtypeStruct((N, D), jnp.float32),
        compiler_params=pltpu.CompilerParams(
            dimension_semantics=("arbitrary",)),
    )(x, agg, agg, W1, b1.reshape(1, D), gamma.reshape(1, D),
      beta.reshape(1, D), W2, b2.reshape(1, D))


def _head_body(h1_ref, h2_ref, h3_ref, b_ref, w1_ref, bb1_ref, w2_ref,
               bb2_ref, o_ref, acc, cnt):
    i = pl.program_id(0)

    @pl.when(i == 0)
    def _init():
        acc[...] = jnp.zeros_like(acc)
        cnt[...] = jnp.zeros_like(cnt)

    rows = lax.broadcasted_iota(jnp.int32, (BR, 1), 0) + i * BR
    valid = rows < N
    gids = lax.broadcasted_iota(jnp.int32, (BR, G), 1)
    onehot = jnp.where((b_ref[...] == gids) & valid, 1.0, 0.0)
    ones = jnp.where(jnp.broadcast_to(valid, (BR, D)), 1.0, 0.0)

    cn = (((0,), (0,)), ((), ()))
    cnt[...] += lax.dot_general(onehot, ones, cn,
                                preferred_element_type=jnp.float32)
    for k, h_ref in enumerate((h1_ref, h2_ref, h3_ref)):
        hm = jnp.where(jnp.broadcast_to(valid, (BR, D)), h_ref[...], 0.0)
        acc[:, k * D:(k + 1) * D] += lax.dot_general(
            onehot, hm, cn, preferred_element_type=jnp.float32)

    @pl.when(i == GRID_R - 1)
    def _final():
        c = jnp.maximum(cnt[...], 1.0)
        pooled = acc[...] / jnp.concatenate([c, c, c], axis=1)
        hh = lax.dot_general(pooled, w1_ref[...], (((1,), (1,)), ((), ())),
                             preferred_element_type=jnp.float32)
        hh = jnp.maximum(hh + bb1_ref[...], 0.0)
        out = lax.dot_general(hh, w2_ref[...], (((1,), (1,)), ((), ())),
                              preferred_element_type=jnp.float32)
        o_ref[...] = out + bb2_ref[...]


def _head(h1, h2, h3, batch2d, lin1_W, lin1_b, lin2_W, lin2_b):
    return pl.pallas_call(
        _head_body,
        grid=(GRID_R,),
        in_specs=[
            pl.BlockSpec((BR, D), lambda i: (i, 0)),
            pl.BlockSpec((BR, D), lambda i: (i, 0)),
            pl.BlockSpec((BR, D), lambda i: (i, 0)),
            pl.BlockSpec((BR, 1), lambda i: (i, 0)),
            pl.BlockSpec((3 * D, 3 * D), lambda i: (0, 0)),
            pl.BlockSpec((1, 3 * D), lambda i: (0, 0)),
            pl.BlockSpec((3, 3 * D), lambda i: (0, 0)),
            pl.BlockSpec((1, 3), lambda i: (0, 0)),
        ],
        out_specs=pl.BlockSpec((G, 3), lambda i: (0, 0)),
        out_shape=jax.ShapeDtypeStruct((G, 3), jnp.float32),
        scratch_shapes=[
            pltpu.VMEM((G, 3 * D), jnp.float32),
            pltpu.VMEM((G, D), jnp.float32),
        ],
        compiler_params=pltpu.CompilerParams(
            dimension_semantics=("arbitrary",)),
    )(h1, h2, h3, batch2d, lin1_W, lin1_b.reshape(1, 3 * D), lin2_W,
      lin2_b.reshape(1, 3))


def kernel(x, edge_index, batch,
           conv1_W1, conv1_b1, conv1_gamma, conv1_beta, conv1_W2, conv1_b2,
           conv2_W1, conv2_b1, conv2_gamma, conv2_beta, conv2_W2, conv2_b2,
           conv3_W1, conv3_b1, conv3_gamma, conv3_beta, conv3_W2, conv3_b2,
           lin1_W, lin1_b, lin2_W, lin2_b):
    src = edge_index[0].astype(jnp.int32)
    dst = edge_index[1].astype(jnp.int32)
    pad = E_PAD - E
    padsrc = jnp.arange(pad, dtype=jnp.int32) % N
    src3 = jnp.concatenate([src, padsrc]).reshape(
        NW, CHUNKS_PER_TILE, CHUNK)
    trash = N + (jnp.arange(pad, dtype=jnp.int32) % (N_PAD - N))
    dst3 = jnp.concatenate([dst, trash]).reshape(
        NW, CHUNKS_PER_TILE, CHUNK)
    zeros = jnp.zeros((CHUNK, D), jnp.float32)
    batch2d = batch.astype(jnp.int32).reshape(N, 1)

    agg1 = _sc_scatter_add(x, src3, dst3, zeros)
    h1 = _mlp(x, agg1, conv1_W1, conv1_b1, conv1_gamma, conv1_beta,
              conv1_W2, conv1_b2)
    agg2 = _sc_scatter_add(h1, src3, dst3, zeros)
    h2 = _mlp(h1, agg2, conv2_W1, conv2_b1, conv2_gamma, conv2_beta,
              conv2_W2, conv2_b2)
    agg3 = _sc_scatter_add(h2, src3, dst3, zeros)
    h3 = _mlp(h2, agg3, conv3_W1, conv3_b1, conv3_gamma, conv3_beta,
              conv3_W2, conv3_b2)
    return _head(h1, h2, h3, batch2d, lin1_W, lin1_b, lin2_W, lin2_b)


# speedup vs baseline: 10.7472x; 1.0017x over previous
import functools

import jax
import jax.numpy as jnp
from jax import lax
from jax.experimental import pallas as pl
from jax.experimental.pallas import tpu as pltpu
from jax.experimental.pallas import tpu_sc as plsc

N = 10000
E = 320000
D = 128
G = 64
BN_EPS = 1e-5

NC, NS = 2, 16
NW = NC * NS
CHUNK = 128
CHUNKS_PER_TILE = 80
HALF_CHUNKS = CHUNKS_PER_TILE // 2
E_PAD = NW * CHUNKS_PER_TILE * CHUNK
N_PAD = 10240
ROWS_PER_TILE = N_PAD // NS
ZCOPIES = ROWS_PER_TILE // CHUNK

BR = 2048
GRID_R = N_PAD // BR


_sc_mesh = plsc.VectorSubcoreMesh(
    core_axis_name="c", subcore_axis_name="s", num_cores=NC, num_subcores=NS)


@functools.partial(
    pl.kernel,
    out_type=jax.ShapeDtypeStruct((NC, N_PAD, D), jnp.float32),
    mesh=_sc_mesh,
    scratch_types=[
        pltpu.VMEM((HALF_CHUNKS, CHUNK), jnp.int32),
        pltpu.VMEM((HALF_CHUNKS, CHUNK), jnp.int32),
        pltpu.VMEM((CHUNK, D), jnp.float32),
        pltpu.VMEM((CHUNK, D), jnp.float32),
        pltpu.VMEM_SHARED((N_PAD, D), jnp.float32),
        pltpu.SemaphoreType.DMA,
        pltpu.SemaphoreType.DMA,
    ],
)
def _sc_scatter_add(x_hbm, src_hbm, dst_hbm, zero_hbm, out_hbm,
                    src_v, dst_v, rows0, rows1, acc_sh, sem0, sem1):
    cid = lax.axis_index("c")
    sid = lax.axis_index("s")
    wid = cid * NS + sid
    row0 = sid * ROWS_PER_TILE

    pltpu.sync_copy(zero_hbm.at[pl.ds(0, CHUNK)], rows0)
    for k in range(ZCOPIES):
        pltpu.sync_copy(rows0, acc_sh.at[pl.ds(row0 + k * CHUNK, CHUNK)])
    plsc.subcore_barrier()

    for half in range(2):
        pltpu.sync_copy(src_hbm.at[wid, pl.ds(half * HALF_CHUNKS, HALF_CHUNKS)],
                        src_v)
        pltpu.sync_copy(dst_hbm.at[wid, pl.ds(half * HALF_CHUNKS, HALF_CHUNKS)],
                        dst_v)
        pltpu.async_copy(x_hbm.at[src_v.at[0]], rows0, sem0)
        pltpu.async_copy(x_hbm.at[src_v.at[1]], rows1, sem1)

        def body(p, carry):
            j0 = 2 * p
            pltpu.make_async_copy(x_hbm.at[src_v.at[0]], rows0, sem0).wait()
            pltpu.sync_copy(rows0, acc_sh.at[dst_v.at[j0]], add=True)
            pltpu.async_copy(x_hbm.at[src_v.at[j0 + 2]], rows0, sem0)
            pltpu.make_async_copy(x_hbm.at[src_v.at[1]], rows1, sem1).wait()
            pltpu.sync_copy(rows1, acc_sh.at[dst_v.at[j0 + 1]], add=True)
            pltpu.async_copy(x_hbm.at[src_v.at[j0 + 3]], rows1, sem1)
            return carry

        lax.fori_loop(0, HALF_CHUNKS // 2 - 1, body, 0)
        pltpu.make_async_copy(x_hbm.at[src_v.at[0]], rows0, sem0).wait()
        pltpu.sync_copy(rows0, acc_sh.at[dst_v.at[HALF_CHUNKS - 2]], add=True)
        pltpu.make_async_copy(x_hbm.at[src_v.at[1]], rows1, sem1).wait()
        pltpu.sync_copy(rows1, acc_sh.at[dst_v.at[HALF_CHUNKS - 1]], add=True)
    plsc.subcore_barrier()

    pltpu.sync_copy(acc_sh.at[pl.ds(row0, ROWS_PER_TILE)],
                    out_hbm.at[cid, pl.ds(row0, ROWS_PER_TILE)])


_INV_SQRT = float(1.0 / (1.0 + BN_EPS) ** 0.5)


def _mlp_body(x_ref, a0_ref, a1_ref, w1_ref, b1_ref, g_ref, be_ref,
              w2_ref, b2_ref, o_ref):
    z = x_ref[...] + a0_ref[0] + a1_ref[0]
    h = lax.dot_general(z, w1_ref[...], (((1,), (1,)), ((), ())),
                        preferred_element_type=jnp.float32)
    h = (h + b1_ref[...]) * (g_ref[...] * _INV_SQRT) + be_ref[...]
    h = jnp.maximum(h, 0.0)
    h = lax.dot_general(h, w2_ref[...], (((1,), (1,)), ((), ())),
                        preferred_element_type=jnp.float32)
    o_ref[...] = jnp.maximum(h + b2_ref[...], 0.0)


def _mlp(x, agg, W1, b1, gamma, beta, W2, b2):
    return pl.pallas_call(
        _mlp_body,
        grid=(GRID_R,),
        in_specs=[
            pl.BlockSpec((BR, D), lambda i: (i, 0)),
            pl.BlockSpec((1, BR, D), lambda i: (0, i, 0)),
            pl.BlockSpec((1, BR, D), lambda i: (1, i, 0)),
            pl.BlockSpec((D, D), lambda i: (0, 0)),
            pl.BlockSpec((1, D), lambda i: (0, 0)),
            pl.BlockSpec((1, D), lambda i: (0, 0)),
            pl.BlockSpec((1, D), lambda i: (0, 0)),
            pl.BlockSpec((D, D), lambda i: (0, 0)),
            pl.BlockSpec((1, D), lambda i: (0, 0)),
        ],
        out_specs=pl.BlockSpec((BR, D), lambda i: (i, 0)),
        out_shape=jax.ShapeDtypeStruct((N, D), jnp.float32),
        compiler_params=pltpu.CompilerParams(
            dimension_semantics=("arbitrary",)),
    )(x, agg, agg, W1, b1.reshape(1, D), gamma.reshape(1, D),
      beta.reshape(1, D), W2, b2.reshape(1, D))


def _pool_body(h_ref, b_ref, s_ref, c_ref, acc, cnt):
    i = pl.program_id(0)

    @pl.when(i == 0)
    def _init():
        acc[...] = jnp.zeros_like(acc)
        cnt[...] = jnp.zeros_like(cnt)

    rows = lax.broadcasted_iota(jnp.int32, (BR, 1), 0) + i * BR
    valid = rows < N
    gids = lax.broadcasted_iota(jnp.int32, (BR, G), 1)
    onehot = jnp.where((b_ref[...] == gids) & valid, 1.0, 0.0)
    ones = jnp.where(jnp.broadcast_to(valid, (BR, D)), 1.0, 0.0)
    hm = jnp.where(jnp.broadcast_to(valid, (BR, D)), h_ref[...], 0.0)

    cn = (((0,), (0,)), ((), ()))
    cnt[...] += lax.dot_general(onehot, ones, cn,
                                preferred_element_type=jnp.float32)
    acc[...] += lax.dot_general(onehot, hm, cn,
                                preferred_element_type=jnp.float32)

    @pl.when(i == GRID_R - 1)
    def _final():
        s_ref[...] = acc[...]
        c_ref[...] = cnt[...]


def _pool(h, batch2d):
    return pl.pallas_call(
        _pool_body,
        grid=(GRID_R,),
        in_specs=[
            pl.BlockSpec((BR, D), lambda i: (i, 0)),
            pl.BlockSpec((BR, 1), lambda i: (i, 0)),
        ],
        out_specs=[
            pl.BlockSpec((G, D), lambda i: (0, 0)),
            pl.BlockSpec((G, D), lambda i: (0, 0)),
        ],
        out_shape=[
            jax.ShapeDtypeStruct((G, D), jnp.float32),
            jax.ShapeDtypeStruct((G, D), jnp.float32),
        ],
        scratch_shapes=[
            pltpu.VMEM((G, D), jnp.float32),
            pltpu.VMEM((G, D), jnp.float32),
        ],
        compiler_params=pltpu.CompilerParams(
            dimension_semantics=("arbitrary",)),
    )(h, batch2d)


def _head_body(s1_ref, s2_ref, s3_ref, c_ref, w1_ref, bb1_ref, w2_ref,
               bb2_ref, o_ref):
    c = jnp.maximum(c_ref[...], 1.0)
    pooled = jnp.concatenate(
        [s1_ref[...] / c, s2_ref[...] / c, s3_ref[...] / c], axis=1)
    hh = lax.dot_general(pooled, w1_ref[...], (((1,), (1,)), ((), ())),
                         preferred_element_type=jnp.float32)
    hh = jnp.maximum(hh + bb1_ref[...], 0.0)
    out = lax.dot_general(hh, w2_ref[...], (((1,), (1,)), ((), ())),
                          preferred_element_type=jnp.float32)
    o_ref[...] = out + bb2_ref[...]


def _head(s1, s2, s3, cnt, lin1_W, lin1_b, lin2_W, lin2_b):
    return pl.pallas_call(
        _head_body,
        out_shape=jax.ShapeDtypeStruct((G, 3), jnp.float32),
    )(s1, s2, s3, cnt, lin1_W, lin1_b.reshape(1, 3 * D), lin2_W,
      lin2_b.reshape(1, 3))


def kernel(x, edge_index, batch,
           conv1_W1, conv1_b1, conv1_gamma, conv1_beta, conv1_W2, conv1_b2,
           conv2_W1, conv2_b1, conv2_gamma, conv2_beta, conv2_W2, conv2_b2,
           conv3_W1, conv3_b1, conv3_gamma, conv3_beta, conv3_W2, conv3_b2,
           lin1_W, lin1_b, lin2_W, lin2_b):
    src = edge_index[0].astype(jnp.int32)
    dst = edge_index[1].astype(jnp.int32)
    pad = E_PAD - E
    padsrc = jnp.arange(pad, dtype=jnp.int32) % N
    src3 = jnp.concatenate([src, padsrc]).reshape(
        NW, CHUNKS_PER_TILE, CHUNK)
    trash = N + (jnp.arange(pad, dtype=jnp.int32) % (N_PAD - N))
    dst3 = jnp.concatenate([dst, trash]).reshape(
        NW, CHUNKS_PER_TILE, CHUNK)
    zeros = jnp.zeros((CHUNK, D), jnp.float32)
    batch2d = batch.astype(jnp.int32).reshape(N, 1)

    agg1 = _sc_scatter_add(x, src3, dst3, zeros)
    h1 = _mlp(x, agg1, conv1_W1, conv1_b1, conv1_gamma, conv1_beta,
              conv1_W2, conv1_b2)
    agg2 = _sc_scatter_add(h1, src3, dst3, zeros)
    s1, cnt = _pool(h1, batch2d)
    h2 = _mlp(h1, agg2, conv2_W1, conv2_b1, conv2_gamma, conv2_beta,
              conv2_W2, conv2_b2)
    agg3 = _sc_scatter_add(h2, src3, dst3, zeros)
    s2, _ = _pool(h2, batch2d)
    h3 = _mlp(h2, agg3, conv3_W1, conv3_b1, conv3_gamma, conv3_beta,
              conv3_W2, conv3_b2)
    s3, _ = _pool(h3, batch2d)
    return _head(s1, s2, s3, cnt, lin1_W, lin1_b, lin2_W, lin2_b)


# speedup vs baseline: 10.8589x; 1.0104x over previous
import functools

import jax
import jax.numpy as jnp
from jax import lax
from jax.experimental import pallas as pl
from jax.experimental.pallas import tpu as pltpu
from jax.experimental.pallas import tpu_sc as plsc

N = 10000
E = 320000
D = 128
G = 64
BN_EPS = 1e-5

NC, NS = 2, 16
NW = NC * NS
CHUNK = 128
CHUNKS_PER_TILE = 80
HALF_CHUNKS = CHUNKS_PER_TILE // 2
E_PAD = NW * CHUNKS_PER_TILE * CHUNK
N_PAD = 10240
ROWS_PER_TILE = N_PAD // NS
ZCOPIES = ROWS_PER_TILE // CHUNK

BR = 2048
GRID_R = N_PAD // BR


_sc_mesh = plsc.VectorSubcoreMesh(
    core_axis_name="c", subcore_axis_name="s", num_cores=NC, num_subcores=NS)


@functools.partial(
    pl.kernel,
    out_type=jax.ShapeDtypeStruct((NC, N_PAD, D), jnp.float32),
    mesh=_sc_mesh,
    scratch_types=[
        pltpu.VMEM((HALF_CHUNKS, CHUNK), jnp.int32),
        pltpu.VMEM((HALF_CHUNKS, CHUNK), jnp.int32),
        pltpu.VMEM((CHUNK, D), jnp.float32),
        pltpu.VMEM((CHUNK, D), jnp.float32),
        pltpu.VMEM_SHARED((N_PAD, D), jnp.float32),
        pltpu.SemaphoreType.DMA,
        pltpu.SemaphoreType.DMA,
    ],
)
def _sc_scatter_add(x_hbm, src_hbm, dst_hbm, zero_hbm, out_hbm,
                    src_v, dst_v, rows0, rows1, acc_sh, sem0, sem1):
    cid = lax.axis_index("c")
    sid = lax.axis_index("s")
    wid = cid * NS + sid
    row0 = sid * ROWS_PER_TILE

    pltpu.sync_copy(zero_hbm.at[pl.ds(0, CHUNK)], rows0)
    for k in range(ZCOPIES):
        pltpu.sync_copy(rows0, acc_sh.at[pl.ds(row0 + k * CHUNK, CHUNK)])
    plsc.subcore_barrier()

    for half in range(2):
        pltpu.sync_copy(src_hbm.at[wid, pl.ds(half * HALF_CHUNKS, HALF_CHUNKS)],
                        src_v)
        pltpu.sync_copy(dst_hbm.at[wid, pl.ds(half * HALF_CHUNKS, HALF_CHUNKS)],
                        dst_v)
        pltpu.async_copy(x_hbm.at[src_v.at[0]], rows0, sem0)
        pltpu.async_copy(x_hbm.at[src_v.at[1]], rows1, sem1)

        def body(p, carry):
            j0 = 2 * p
            pltpu.make_async_copy(x_hbm.at[src_v.at[0]], rows0, sem0).wait()
            pltpu.sync_copy(rows0, acc_sh.at[dst_v.at[j0]], add=True)
            pltpu.async_copy(x_hbm.at[src_v.at[j0 + 2]], rows0, sem0)
            pltpu.make_async_copy(x_hbm.at[src_v.at[1]], rows1, sem1).wait()
            pltpu.sync_copy(rows1, acc_sh.at[dst_v.at[j0 + 1]], add=True)
            pltpu.async_copy(x_hbm.at[src_v.at[j0 + 3]], rows1, sem1)
            return carry

        lax.fori_loop(0, HALF_CHUNKS // 2 - 1, body, 0)
        pltpu.make_async_copy(x_hbm.at[src_v.at[0]], rows0, sem0).wait()
        pltpu.sync_copy(rows0, acc_sh.at[dst_v.at[HALF_CHUNKS - 2]], add=True)
        pltpu.make_async_copy(x_hbm.at[src_v.at[1]], rows1, sem1).wait()
        pltpu.sync_copy(rows1, acc_sh.at[dst_v.at[HALF_CHUNKS - 1]], add=True)
    plsc.subcore_barrier()

    pltpu.sync_copy(acc_sh.at[pl.ds(row0, ROWS_PER_TILE)],
                    out_hbm.at[cid, pl.ds(row0, ROWS_PER_TILE)])


_INV_SQRT = float(1.0 / (1.0 + BN_EPS) ** 0.5)


def _mlp_body(x_ref, a0_ref, a1_ref, b_ref, w1_ref, b1_ref, g_ref, be_ref,
              w2_ref, b2_ref, o_ref, s_ref, c_ref, acc, cnt):
    i = pl.program_id(0)

    @pl.when(i == 0)
    def _init():
        acc[...] = jnp.zeros_like(acc)
        cnt[...] = jnp.zeros_like(cnt)

    z = x_ref[...] + a0_ref[0] + a1_ref[0]
    h = lax.dot_general(z, w1_ref[...], (((1,), (1,)), ((), ())),
                        preferred_element_type=jnp.float32)
    h = (h + b1_ref[...]) * (g_ref[...] * _INV_SQRT) + be_ref[...]
    h = jnp.maximum(h, 0.0)
    h = lax.dot_general(h, w2_ref[...], (((1,), (1,)), ((), ())),
                        preferred_element_type=jnp.float32)
    h = jnp.maximum(h + b2_ref[...], 0.0)
    o_ref[...] = h

    rows = lax.broadcasted_iota(jnp.int32, (BR, 1), 0) + i * BR
    valid = rows < N
    gids = lax.broadcasted_iota(jnp.int32, (BR, G), 1)
    onehot = jnp.where((b_ref[...] == gids) & valid, 1.0, 0.0)
    ones = jnp.where(jnp.broadcast_to(valid, (BR, D)), 1.0, 0.0)
    hm = jnp.where(jnp.broadcast_to(valid, (BR, D)), h, 0.0)

    cn = (((0,), (0,)), ((), ()))
    cnt[...] += lax.dot_general(onehot, ones, cn,
                                preferred_element_type=jnp.float32)
    acc[...] += lax.dot_general(onehot, hm, cn,
                                preferred_element_type=jnp.float32)

    @pl.when(i == GRID_R - 1)
    def _final():
        s_ref[...] = acc[...]
        c_ref[...] = cnt[...]


def _mlp(x, agg, batch2d, W1, b1, gamma, beta, W2, b2):
    return pl.pallas_call(
        _mlp_body,
        grid=(GRID_R,),
        in_specs=[
            pl.BlockSpec((BR, D), lambda i: (i, 0)),
            pl.BlockSpec((1, BR, D), lambda i: (0, i, 0)),
            pl.BlockSpec((1, BR, D), lambda i: (1, i, 0)),
            pl.BlockSpec((BR, 1), lambda i: (i, 0)),
            pl.BlockSpec((D, D), lambda i: (0, 0)),
            pl.BlockSpec((1, D), lambda i: (0, 0)),
            pl.BlockSpec((1, D), lambda i: (0, 0)),
            pl.BlockSpec((1, D), lambda i: (0, 0)),
            pl.BlockSpec((D, D), lambda i: (0, 0)),
            pl.BlockSpec((1, D), lambda i: (0, 0)),
        ],
        out_specs=[
            pl.BlockSpec((BR, D), lambda i: (i, 0)),
            pl.BlockSpec((G, D), lambda i: (0, 0)),
            pl.BlockSpec((G, D), lambda i: (0, 0)),
        ],
        out_shape=[
            jax.ShapeDtypeStruct((N, D), jnp.float32),
            jax.ShapeDtypeStruct((G, D), jnp.float32),
            jax.ShapeDtypeStruct((G, D), jnp.float32),
        ],
        scratch_shapes=[
            pltpu.VMEM((G, D), jnp.float32),
            pltpu.VMEM((G, D), jnp.float32),
        ],
        compiler_params=pltpu.CompilerParams(
            dimension_semantics=("arbitrary",)),
    )(x, agg, agg, batch2d, W1, b1.reshape(1, D), gamma.reshape(1, D),
      beta.reshape(1, D), W2, b2.reshape(1, D))


def _head_body(s1_ref, s2_ref, s3_ref, c_ref, w1_ref, bb1_ref, w2_ref,
               bb2_ref, o_ref):
    c = jnp.maximum(c_ref[...], 1.0)
    pooled = jnp.concatenate(
        [s1_ref[...] / c, s2_ref[...] / c, s3_ref[...] / c], axis=1)
    hh = lax.dot_general(pooled, w1_ref[...], (((1,), (1,)), ((), ())),
                         preferred_element_type=jnp.float32)
    hh = jnp.maximum(hh + bb1_ref[...], 0.0)
    out = lax.dot_general(hh, w2_ref[...], (((1,), (1,)), ((), ())),
                          preferred_element_type=jnp.float32)
    o_ref[...] = out + bb2_ref[...]


def _head(s1, s2, s3, cnt, lin1_W, lin1_b, lin2_W, lin2_b):
    return pl.pallas_call(
        _head_body,
        out_shape=jax.ShapeDtypeStruct((G, 3), jnp.float32),
    )(s1, s2, s3, cnt, lin1_W, lin1_b.reshape(1, 3 * D), lin2_W,
      lin2_b.reshape(1, 3))


def kernel(x, edge_index, batch,
           conv1_W1, conv1_b1, conv1_gamma, conv1_beta, conv1_W2, conv1_b2,
           conv2_W1, conv2_b1, conv2_gamma, conv2_beta, conv2_W2, conv2_b2,
           conv3_W1, conv3_b1, conv3_gamma, conv3_beta, conv3_W2, conv3_b2,
           lin1_W, lin1_b, lin2_W, lin2_b):
    src = edge_index[0].astype(jnp.int32)
    dst = edge_index[1].astype(jnp.int32)
    pad = E_PAD - E
    padsrc = jnp.arange(pad, dtype=jnp.int32) % N
    src3 = jnp.concatenate([src, padsrc]).reshape(
        NW, CHUNKS_PER_TILE, CHUNK)
    trash = N + (jnp.arange(pad, dtype=jnp.int32) % (N_PAD - N))
    dst3 = jnp.concatenate([dst, trash]).reshape(
        NW, CHUNKS_PER_TILE, CHUNK)
    zeros = jnp.zeros((CHUNK, D), jnp.float32)
    batch2d = batch.astype(jnp.int32).reshape(N, 1)

    agg1 = _sc_scatter_add(x, src3, dst3, zeros)
    h1, s1, cnt = _mlp(x, agg1, batch2d, conv1_W1, conv1_b1, conv1_gamma,
                       conv1_beta, conv1_W2, conv1_b2)
    agg2 = _sc_scatter_add(h1, src3, dst3, zeros)
    h2, s2, _ = _mlp(h1, agg2, batch2d, conv2_W1, conv2_b1, conv2_gamma,
                     conv2_beta, conv2_W2, conv2_b2)
    agg3 = _sc_scatter_add(h2, src3, dst3, zeros)
    h3, s3, _ = _mlp(h2, agg3, batch2d, conv3_W1, conv3_b1, conv3_gamma,
                     conv3_beta, conv3_W2, conv3_b2)
    return _head(s1, s2, s3, cnt, lin1_W, lin1_b, lin2_W, lin2_b)


# speedup vs baseline: 11.0428x; 1.0169x over previous
import functools

import jax
import jax.numpy as jnp
from jax import lax
from jax.experimental import pallas as pl
from jax.experimental.pallas import tpu as pltpu
from jax.experimental.pallas import tpu_sc as plsc

N = 10000
E = 320000
D = 128
G = 64
BN_EPS = 1e-5

NC, NS = 2, 16
NW = NC * NS
CHUNK = 128
CHUNKS_PER_TILE = 80
HALF_CHUNKS = CHUNKS_PER_TILE // 2
E_PAD = NW * CHUNKS_PER_TILE * CHUNK
N_PAD = 10240
ROWS_PER_TILE = N_PAD // NS
ZCOPIES = ROWS_PER_TILE // CHUNK

BR = 2048
GRID_R = N_PAD // BR


_sc_mesh = plsc.VectorSubcoreMesh(
    core_axis_name="c", subcore_axis_name="s", num_cores=NC, num_subcores=NS)


@functools.partial(
    pl.kernel,
    out_type=jax.ShapeDtypeStruct((NC, N_PAD, D), jnp.float32),
    mesh=_sc_mesh,
    scratch_types=[
        pltpu.VMEM((HALF_CHUNKS, CHUNK), jnp.int32),
        pltpu.VMEM((HALF_CHUNKS, CHUNK), jnp.int32),
        pltpu.VMEM((CHUNK, D), jnp.float32),
        pltpu.VMEM((CHUNK, D), jnp.float32),
        pltpu.VMEM_SHARED((N_PAD, D), jnp.float32),
        pltpu.SemaphoreType.DMA,
        pltpu.SemaphoreType.DMA,
    ],
)
def _sc_scatter_add(x_hbm, src_hbm, dst_hbm, zero_hbm, out_hbm,
                    src_v, dst_v, rows0, rows1, acc_sh, sem0, sem1):
    cid = lax.axis_index("c")
    sid = lax.axis_index("s")
    wid = cid * NS + sid
    row0 = sid * ROWS_PER_TILE

    pltpu.sync_copy(zero_hbm.at[pl.ds(0, CHUNK)], rows0)
    for k in range(ZCOPIES):
        pltpu.sync_copy(rows0, acc_sh.at[pl.ds(row0 + k * CHUNK, CHUNK)])
    plsc.subcore_barrier()

    for half in range(2):
        pltpu.sync_copy(src_hbm.at[wid, pl.ds(half * HALF_CHUNKS, HALF_CHUNKS)],
                        src_v)
        pltpu.sync_copy(dst_hbm.at[wid, pl.ds(half * HALF_CHUNKS, HALF_CHUNKS)],
                        dst_v)
        pltpu.async_copy(x_hbm.at[src_v.at[0]], rows0, sem0)
        pltpu.async_copy(x_hbm.at[src_v.at[1]], rows1, sem1)

        def body(p, carry):
            j0 = 2 * p
            pltpu.make_async_copy(x_hbm.at[src_v.at[0]], rows0, sem0).wait()
            pltpu.sync_copy(rows0, acc_sh.at[dst_v.at[j0]], add=True)
            pltpu.async_copy(x_hbm.at[src_v.at[j0 + 2]], rows0, sem0)
            pltpu.make_async_copy(x_hbm.at[src_v.at[1]], rows1, sem1).wait()
            pltpu.sync_copy(rows1, acc_sh.at[dst_v.at[j0 + 1]], add=True)
            pltpu.async_copy(x_hbm.at[src_v.at[j0 + 3]], rows1, sem1)
            return carry

        lax.fori_loop(0, HALF_CHUNKS // 2 - 1, body, 0)
        pltpu.make_async_copy(x_hbm.at[src_v.at[0]], rows0, sem0).wait()
        pltpu.sync_copy(rows0, acc_sh.at[dst_v.at[HALF_CHUNKS - 2]], add=True)
        pltpu.make_async_copy(x_hbm.at[src_v.at[1]], rows1, sem1).wait()
        pltpu.sync_copy(rows1, acc_sh.at[dst_v.at[HALF_CHUNKS - 1]], add=True)
    plsc.subcore_barrier()

    pltpu.sync_copy(acc_sh.at[pl.ds(row0, ROWS_PER_TILE)],
                    out_hbm.at[cid, pl.ds(row0, ROWS_PER_TILE)])


_INV_SQRT = float(1.0 / (1.0 + BN_EPS) ** 0.5)


def _mlp_body(x_ref, a0_ref, a1_ref, b_ref, w1_ref, b1_ref, g_ref, be_ref,
              w2_ref, b2_ref, o_ref, s_ref, c_ref, acc, cnt):
    i = pl.program_id(0)

    @pl.when(i == 0)
    def _init():
        acc[...] = jnp.zeros_like(acc)
        cnt[...] = jnp.zeros_like(cnt)

    z = x_ref[...] + a0_ref[0] + a1_ref[0]
    h = lax.dot_general(z, w1_ref[...], (((1,), (1,)), ((), ())),
                        preferred_element_type=jnp.float32)
    h = (h + b1_ref[...]) * (g_ref[...] * _INV_SQRT) + be_ref[...]
    h = jnp.maximum(h, 0.0)
    h = lax.dot_general(h, w2_ref[...], (((1,), (1,)), ((), ())),
                        preferred_element_type=jnp.float32)
    h = jnp.maximum(h + b2_ref[...], 0.0)
    o_ref[...] = h

    rows = lax.broadcasted_iota(jnp.int32, (BR, 1), 0) + i * BR
    valid = rows < N
    gids = lax.broadcasted_iota(jnp.int32, (BR, G), 1)
    onehot = jnp.where((b_ref[...] == gids) & valid, 1.0, 0.0)
    ones = jnp.where(jnp.broadcast_to(valid, (BR, D)), 1.0, 0.0)
    hm = jnp.where(jnp.broadcast_to(valid, (BR, D)), h, 0.0)

    cn = (((0,), (0,)), ((), ()))
    cnt[...] += lax.dot_general(onehot, ones, cn,
                                preferred_element_type=jnp.float32)
    acc[...] += lax.dot_general(onehot, hm, cn,
                                preferred_element_type=jnp.float32)

    @pl.when(i == GRID_R - 1)
    def _final():
        s_ref[...] = acc[...]
        c_ref[...] = cnt[...]


def _mlp(x, agg, batch2d, W1, b1, gamma, beta, W2, b2):
    return pl.pallas_call(
        _mlp_body,
        grid=(GRID_R,),
        in_specs=[
            pl.BlockSpec((BR, D), lambda i: (i, 0)),
            pl.BlockSpec((1, BR, D), lambda i: (0, i, 0)),
            pl.BlockSpec((1, BR, D), lambda i: (1, i, 0)),
            pl.BlockSpec((BR, 1), lambda i: (i, 0)),
            pl.BlockSpec((D, D), lambda i: (0, 0)),
            pl.BlockSpec((1, D), lambda i: (0, 0)),
            pl.BlockSpec((1, D), lambda i: (0, 0)),
            pl.BlockSpec((1, D), lambda i: (0, 0)),
            pl.BlockSpec((D, D), lambda i: (0, 0)),
            pl.BlockSpec((1, D), lambda i: (0, 0)),
        ],
        out_specs=[
            pl.BlockSpec((BR, D), lambda i: (i, 0)),
            pl.BlockSpec((G, D), lambda i: (0, 0)),
            pl.BlockSpec((G, D), lambda i: (0, 0)),
        ],
        out_shape=[
            jax.ShapeDtypeStruct((N, D), jnp.float32),
            jax.ShapeDtypeStruct((G, D), jnp.float32),
            jax.ShapeDtypeStruct((G, D), jnp.float32),
        ],
        scratch_shapes=[
            pltpu.VMEM((G, D), jnp.float32),
            pltpu.VMEM((G, D), jnp.float32),
        ],
        compiler_params=pltpu.CompilerParams(
            dimension_semantics=("arbitrary",)),
    )(x, agg, agg, batch2d, W1, b1.reshape(1, D), gamma.reshape(1, D),
      beta.reshape(1, D), W2, b2.reshape(1, D))


EB = E // CHUNK
EPB = E_PAD // CHUNK
PB = EPB - EB


def _prep_body(e_ref, s_ref, d_ref):
    s_ref[:EB] = e_ref[0]
    d_ref[:EB] = e_ref[1]
    ii = (lax.broadcasted_iota(jnp.int32, (PB, CHUNK), 0) * CHUNK +
          lax.broadcasted_iota(jnp.int32, (PB, CHUNK), 1))
    s_ref[EB:] = lax.rem(ii, N)
    d_ref[EB:] = N + lax.rem(ii, N_PAD - N)


def _prep(e32):
    return pl.pallas_call(
        _prep_body,
        out_shape=[
            jax.ShapeDtypeStruct((EPB, CHUNK), jnp.int32),
            jax.ShapeDtypeStruct((EPB, CHUNK), jnp.int32),
        ],
    )(e32)


def _head_body(s1_ref, s2_ref, s3_ref, c_ref, w1_ref, bb1_ref, w2_ref,
               bb2_ref, o_ref):
    c = jnp.maximum(c_ref[...], 1.0)
    pooled = jnp.concatenate(
        [s1_ref[...] / c, s2_ref[...] / c, s3_ref[...] / c], axis=1)
    hh = lax.dot_general(pooled, w1_ref[...], (((1,), (1,)), ((), ())),
                         preferred_element_type=jnp.float32)
    hh = jnp.maximum(hh + bb1_ref[...], 0.0)
    out = lax.dot_general(hh, w2_ref[...], (((1,), (1,)), ((), ())),
                          preferred_element_type=jnp.float32)
    o_ref[...] = out + bb2_ref[...]


def _head(s1, s2, s3, cnt, lin1_W, lin1_b, lin2_W, lin2_b):
    return pl.pallas_call(
        _head_body,
        out_shape=jax.ShapeDtypeStruct((G, 3), jnp.float32),
    )(s1, s2, s3, cnt, lin1_W, lin1_b.reshape(1, 3 * D), lin2_W,
      lin2_b.reshape(1, 3))


def kernel(x, edge_index, batch,
           conv1_W1, conv1_b1, conv1_gamma, conv1_beta, conv1_W2, conv1_b2,
           conv2_W1, conv2_b1, conv2_gamma, conv2_beta, conv2_W2, conv2_b2,
           conv3_W1, conv3_b1, conv3_gamma, conv3_beta, conv3_W2, conv3_b2,
           lin1_W, lin1_b, lin2_W, lin2_b):
    e32 = edge_index.astype(jnp.int32).reshape(2, EB, CHUNK)
    src2, dst2 = _prep(e32)
    src3 = src2.reshape(NW, CHUNKS_PER_TILE, CHUNK)
    dst3 = dst2.reshape(NW, CHUNKS_PER_TILE, CHUNK)
    zeros = jnp.zeros((CHUNK, D), jnp.float32)
    batch2d = batch.astype(jnp.int32).reshape(N, 1)

    agg1 = _sc_scatter_add(x, src3, dst3, zeros)
    h1, s1, cnt = _mlp(x, agg1, batch2d, conv1_W1, conv1_b1, conv1_gamma,
                       conv1_beta, conv1_W2, conv1_b2)
    agg2 = _sc_scatter_add(h1, src3, dst3, zeros)
    h2, s2, _ = _mlp(h1, agg2, batch2d, conv2_W1, conv2_b1, conv2_gamma,
                     conv2_beta, conv2_W2, conv2_b2)
    agg3 = _sc_scatter_add(h2, src3, dst3, zeros)
    h3, s3, _ = _mlp(h2, agg3, batch2d, conv3_W1, conv3_b1, conv3_gamma,
                     conv3_beta, conv3_W2, conv3_b2)
    return _head(s1, s2, s3, cnt, lin1_W, lin1_b, lin2_W, lin2_b)


# speedup vs baseline: 11.1326x; 1.0081x over previous
import functools

import jax
import jax.numpy as jnp
from jax import lax
from jax.experimental import pallas as pl
from jax.experimental.pallas import tpu as pltpu
from jax.experimental.pallas import tpu_sc as plsc

N = 10000
E = 320000
D = 128
G = 64
BN_EPS = 1e-5

NC, NS = 2, 16
NW = NC * NS
CHUNK = 128
CHUNKS_PER_TILE = 80
HALF_CHUNKS = CHUNKS_PER_TILE // 2
E_PAD = NW * CHUNKS_PER_TILE * CHUNK
N_PAD = 10240
ROWS_PER_TILE = N_PAD // NS
ZCOPIES = ROWS_PER_TILE // CHUNK

BR = 5120
GRID_R = N_PAD // BR


_sc_mesh = plsc.VectorSubcoreMesh(
    core_axis_name="c", subcore_axis_name="s", num_cores=NC, num_subcores=NS)


@functools.partial(
    pl.kernel,
    out_type=jax.ShapeDtypeStruct((NC, N_PAD, D), jnp.float32),
    mesh=_sc_mesh,
    scratch_types=[
        pltpu.VMEM((HALF_CHUNKS, CHUNK), jnp.int32),
        pltpu.VMEM((HALF_CHUNKS, CHUNK), jnp.int32),
        pltpu.VMEM((CHUNK, D), jnp.float32),
        pltpu.VMEM((CHUNK, D), jnp.float32),
        pltpu.VMEM_SHARED((N_PAD, D), jnp.float32),
        pltpu.SemaphoreType.DMA,
        pltpu.SemaphoreType.DMA,
    ],
)
def _sc_scatter_add(x_hbm, src_hbm, dst_hbm, zero_hbm, out_hbm,
                    src_v, dst_v, rows0, rows1, acc_sh, sem0, sem1):
    cid = lax.axis_index("c")
    sid = lax.axis_index("s")
    wid = cid * NS + sid
    row0 = sid * ROWS_PER_TILE

    pltpu.sync_copy(zero_hbm.at[pl.ds(0, CHUNK)], rows0)
    for k in range(ZCOPIES):
        pltpu.sync_copy(rows0, acc_sh.at[pl.ds(row0 + k * CHUNK, CHUNK)])
    plsc.subcore_barrier()

    for half in range(2):
        pltpu.sync_copy(src_hbm.at[wid, pl.ds(half * HALF_CHUNKS, HALF_CHUNKS)],
                        src_v)
        pltpu.sync_copy(dst_hbm.at[wid, pl.ds(half * HALF_CHUNKS, HALF_CHUNKS)],
                        dst_v)
        pltpu.async_copy(x_hbm.at[src_v.at[0]], rows0, sem0)
        pltpu.async_copy(x_hbm.at[src_v.at[1]], rows1, sem1)

        def body(p, carry):
            j0 = 2 * p
            pltpu.make_async_copy(x_hbm.at[src_v.at[0]], rows0, sem0).wait()
            pltpu.sync_copy(rows0, acc_sh.at[dst_v.at[j0]], add=True)
            pltpu.async_copy(x_hbm.at[src_v.at[j0 + 2]], rows0, sem0)
            pltpu.make_async_copy(x_hbm.at[src_v.at[1]], rows1, sem1).wait()
            pltpu.sync_copy(rows1, acc_sh.at[dst_v.at[j0 + 1]], add=True)
            pltpu.async_copy(x_hbm.at[src_v.at[j0 + 3]], rows1, sem1)
            return carry

        lax.fori_loop(0, HALF_CHUNKS // 2 - 1, body, 0)
        pltpu.make_async_copy(x_hbm.at[src_v.at[0]], rows0, sem0).wait()
        pltpu.sync_copy(rows0, acc_sh.at[dst_v.at[HALF_CHUNKS - 2]], add=True)
        pltpu.make_async_copy(x_hbm.at[src_v.at[1]], rows1, sem1).wait()
        pltpu.sync_copy(rows1, acc_sh.at[dst_v.at[HALF_CHUNKS - 1]], add=True)
    plsc.subcore_barrier()

    pltpu.sync_copy(acc_sh.at[pl.ds(row0, ROWS_PER_TILE)],
                    out_hbm.at[cid, pl.ds(row0, ROWS_PER_TILE)])


_INV_SQRT = float(1.0 / (1.0 + BN_EPS) ** 0.5)


def _mlp_body(x_ref, a0_ref, a1_ref, b_ref, w1_ref, b1_ref, g_ref, be_ref,
              w2_ref, b2_ref, o_ref, s_ref, c_ref, acc, cnt):
    i = pl.program_id(0)

    @pl.when(i == 0)
    def _init():
        acc[...] = jnp.zeros_like(acc)
        cnt[...] = jnp.zeros_like(cnt)

    z = x_ref[...] + a0_ref[0] + a1_ref[0]
    h = lax.dot_general(z, w1_ref[...], (((1,), (1,)), ((), ())),
                        preferred_element_type=jnp.float32)
    h = (h + b1_ref[...]) * (g_ref[...] * _INV_SQRT) + be_ref[...]
    h = jnp.maximum(h, 0.0)
    h = lax.dot_general(h, w2_ref[...], (((1,), (1,)), ((), ())),
                        preferred_element_type=jnp.float32)
    h = jnp.maximum(h + b2_ref[...], 0.0)
    o_ref[...] = h

    rows = lax.broadcasted_iota(jnp.int32, (BR, 1), 0) + i * BR
    valid = rows < N
    gids = lax.broadcasted_iota(jnp.int32, (BR, G), 1)
    onehot = jnp.where((b_ref[...] == gids) & valid, 1.0, 0.0)
    ones = jnp.where(jnp.broadcast_to(valid, (BR, D)), 1.0, 0.0)
    hm = jnp.where(jnp.broadcast_to(valid, (BR, D)), h, 0.0)

    cn = (((0,), (0,)), ((), ()))
    cnt[...] += lax.dot_general(onehot, ones, cn,
                                preferred_element_type=jnp.float32)
    acc[...] += lax.dot_general(onehot, hm, cn,
                                preferred_element_type=jnp.float32)

    @pl.when(i == GRID_R - 1)
    def _final():
        s_ref[...] = acc[...]
        c_ref[...] = cnt[...]


def _mlp(x, agg, batch2d, W1, b1, gamma, beta, W2, b2):
    return pl.pallas_call(
        _mlp_body,
        grid=(GRID_R,),
        in_specs=[
            pl.BlockSpec((BR, D), lambda i: (i, 0)),
            pl.BlockSpec((1, BR, D), lambda i: (0, i, 0)),
            pl.BlockSpec((1, BR, D), lambda i: (1, i, 0)),
            pl.BlockSpec((BR, 1), lambda i: (i, 0)),
            pl.BlockSpec((D, D), lambda i: (0, 0)),
            pl.BlockSpec((1, D), lambda i: (0, 0)),
            pl.BlockSpec((1, D), lambda i: (0, 0)),
            pl.BlockSpec((1, D), lambda i: (0, 0)),
            pl.BlockSpec((D, D), lambda i: (0, 0)),
            pl.BlockSpec((1, D), lambda i: (0, 0)),
        ],
        out_specs=[
            pl.BlockSpec((BR, D), lambda i: (i, 0)),
            pl.BlockSpec((G, D), lambda i: (0, 0)),
            pl.BlockSpec((G, D), lambda i: (0, 0)),
        ],
        out_shape=[
            jax.ShapeDtypeStruct((N, D), jnp.float32),
            jax.ShapeDtypeStruct((G, D), jnp.float32),
            jax.ShapeDtypeStruct((G, D), jnp.float32),
        ],
        scratch_shapes=[
            pltpu.VMEM((G, D), jnp.float32),
            pltpu.VMEM((G, D), jnp.float32),
        ],
        compiler_params=pltpu.CompilerParams(
            dimension_semantics=("arbitrary",)),
    )(x, agg, agg, batch2d, W1, b1.reshape(1, D), gamma.reshape(1, D),
      beta.reshape(1, D), W2, b2.reshape(1, D))


EB = E // CHUNK
EPB = E_PAD // CHUNK
PB = EPB - EB


def _prep_body(e_ref, s_ref, d_ref):
    s_ref[:EB] = e_ref[0]
    d_ref[:EB] = e_ref[1]
    ii = (lax.broadcasted_iota(jnp.int32, (PB, CHUNK), 0) * CHUNK +
          lax.broadcasted_iota(jnp.int32, (PB, CHUNK), 1))
    s_ref[EB:] = lax.rem(ii, N)
    d_ref[EB:] = N + lax.rem(ii, N_PAD - N)


def _prep(e32):
    return pl.pallas_call(
        _prep_body,
        out_shape=[
            jax.ShapeDtypeStruct((EPB, CHUNK), jnp.int32),
            jax.ShapeDtypeStruct((EPB, CHUNK), jnp.int32),
        ],
    )(e32)


def _head_body(s1_ref, s2_ref, s3_ref, c_ref, w1_ref, bb1_ref, w2_ref,
               bb2_ref, o_ref):
    c = jnp.maximum(c_ref[...], 1.0)
    pooled = jnp.concatenate(
        [s1_ref[...] / c, s2_ref[...] / c, s3_ref[...] / c], axis=1)
    hh = lax.dot_general(pooled, w1_ref[...], (((1,), (1,)), ((), ())),
                         preferred_element_type=jnp.float32)
    hh = jnp.maximum(hh + bb1_ref[...], 0.0)
    out = lax.dot_general(hh, w2_ref[...], (((1,), (1,)), ((), ())),
                          preferred_element_type=jnp.float32)
    o_ref[...] = out + bb2_ref[...]


def _head(s1, s2, s3, cnt, lin1_W, lin1_b, lin2_W, lin2_b):
    return pl.pallas_call(
        _head_body,
        out_shape=jax.ShapeDtypeStruct((G, 3), jnp.float32),
    )(s1, s2, s3, cnt, lin1_W, lin1_b.reshape(1, 3 * D), lin2_W,
      lin2_b.reshape(1, 3))


def kernel(x, edge_index, batch,
           conv1_W1, conv1_b1, conv1_gamma, conv1_beta, conv1_W2, conv1_b2,
           conv2_W1, conv2_b1, conv2_gamma, conv2_beta, conv2_W2, conv2_b2,
           conv3_W1, conv3_b1, conv3_gamma, conv3_beta, conv3_W2, conv3_b2,
           lin1_W, lin1_b, lin2_W, lin2_b):
    e32 = edge_index.astype(jnp.int32).reshape(2, EB, CHUNK)
    src2, dst2 = _prep(e32)
    src3 = src2.reshape(NW, CHUNKS_PER_TILE, CHUNK)
    dst3 = dst2.reshape(NW, CHUNKS_PER_TILE, CHUNK)
    zeros = jnp.zeros((CHUNK, D), jnp.float32)
    batch2d = batch.astype(jnp.int32).reshape(N, 1)

    agg1 = _sc_scatter_add(x, src3, dst3, zeros)
    h1, s1, cnt = _mlp(x, agg1, batch2d, conv1_W1, conv1_b1, conv1_gamma,
                       conv1_beta, conv1_W2, conv1_b2)
    agg2 = _sc_scatter_add(h1, src3, dst3, zeros)
    h2, s2, _ = _mlp(h1, agg2, batch2d, conv2_W1, conv2_b1, conv2_gamma,
                     conv2_beta, conv2_W2, conv2_b2)
    agg3 = _sc_scatter_add(h2, src3, dst3, zeros)
    h3, s3, _ = _mlp(h2, agg3, batch2d, conv3_W1, conv3_b1, conv3_gamma,
                     conv3_beta, conv3_W2, conv3_b2)
    return _head(s1, s2, s3, cnt, lin1_W, lin1_b, lin2_W, lin2_b)


# speedup vs baseline: 11.4460x; 1.0282x over previous
import functools

import jax
import jax.numpy as jnp
from jax import lax
from jax.experimental import pallas as pl
from jax.experimental.pallas import tpu as pltpu
from jax.experimental.pallas import tpu_sc as plsc

N = 10000
E = 320000
D = 128
G = 64
BN_EPS = 1e-5

NC, NS = 2, 16
NW = NC * NS
CHUNK = 128
CHUNKS_PER_TILE = 80
HALF_CHUNKS = CHUNKS_PER_TILE // 2
E_PAD = NW * CHUNKS_PER_TILE * CHUNK
N_PAD = 10240
ROWS_PER_TILE = N_PAD // NS
ZCOPIES = ROWS_PER_TILE // CHUNK

BR = 5120
GRID_R = N_PAD // BR


_sc_mesh = plsc.VectorSubcoreMesh(
    core_axis_name="c", subcore_axis_name="s", num_cores=NC, num_subcores=NS)


@functools.partial(
    pl.kernel,
    out_type=jax.ShapeDtypeStruct((NC, N_PAD, D), jnp.float32),
    mesh=_sc_mesh,
    scratch_types=[
        pltpu.VMEM((HALF_CHUNKS, CHUNK), jnp.int32),
        pltpu.VMEM((HALF_CHUNKS, CHUNK), jnp.int32),
        pltpu.VMEM((CHUNK, D), jnp.float32),
        pltpu.VMEM((CHUNK, D), jnp.float32),
        pltpu.VMEM_SHARED((N_PAD, D), jnp.float32),
        pltpu.SemaphoreType.DMA,
        pltpu.SemaphoreType.DMA,
        pltpu.SemaphoreType.DMA,
    ],
)
def _sc_scatter_add(x_hbm, src_hbm, dst_hbm, zero_hbm, out_hbm,
                    src_v, dst_v, rows0, rows1, acc_sh, sem0, sem1, semi):
    cid = lax.axis_index("c")
    sid = lax.axis_index("s")
    wid = cid * NS + sid
    row0 = sid * ROWS_PER_TILE

    pltpu.async_copy(src_hbm.at[wid, pl.ds(0, HALF_CHUNKS)], src_v, semi)
    pltpu.async_copy(dst_hbm.at[wid, pl.ds(0, HALF_CHUNKS)], dst_v, semi)

    pltpu.sync_copy(zero_hbm.at[pl.ds(sid * CHUNK, CHUNK)], rows0)
    for k in range(ZCOPIES):
        pltpu.sync_copy(rows0, acc_sh.at[pl.ds(row0 + k * CHUNK, CHUNK)])
    pltpu.make_async_copy(src_hbm.at[wid, pl.ds(0, HALF_CHUNKS)], src_v,
                          semi).wait()
    pltpu.make_async_copy(dst_hbm.at[wid, pl.ds(0, HALF_CHUNKS)], dst_v,
                          semi).wait()
    plsc.subcore_barrier()

    for half in range(2):
        if half:
            pltpu.sync_copy(
                src_hbm.at[wid, pl.ds(half * HALF_CHUNKS, HALF_CHUNKS)],
                src_v)
            pltpu.sync_copy(
                dst_hbm.at[wid, pl.ds(half * HALF_CHUNKS, HALF_CHUNKS)],
                dst_v)
        pltpu.async_copy(x_hbm.at[src_v.at[0]], rows0, sem0)
        pltpu.async_copy(x_hbm.at[src_v.at[1]], rows1, sem1)

        def body(p, carry):
            j0 = 2 * p
            pltpu.make_async_copy(x_hbm.at[src_v.at[0]], rows0, sem0).wait()
            pltpu.sync_copy(rows0, acc_sh.at[dst_v.at[j0]], add=True)
            pltpu.async_copy(x_hbm.at[src_v.at[j0 + 2]], rows0, sem0)
            pltpu.make_async_copy(x_hbm.at[src_v.at[1]], rows1, sem1).wait()
            pltpu.sync_copy(rows1, acc_sh.at[dst_v.at[j0 + 1]], add=True)
            pltpu.async_copy(x_hbm.at[src_v.at[j0 + 3]], rows1, sem1)
            return carry

        lax.fori_loop(0, HALF_CHUNKS // 2 - 1, body, 0)
        pltpu.make_async_copy(x_hbm.at[src_v.at[0]], rows0, sem0).wait()
        pltpu.sync_copy(rows0, acc_sh.at[dst_v.at[HALF_CHUNKS - 2]], add=True)
        pltpu.make_async_copy(x_hbm.at[src_v.at[1]], rows1, sem1).wait()
        pltpu.sync_copy(rows1, acc_sh.at[dst_v.at[HALF_CHUNKS - 1]], add=True)
    plsc.subcore_barrier()

    pltpu.sync_copy(acc_sh.at[pl.ds(row0, ROWS_PER_TILE)],
                    out_hbm.at[cid, pl.ds(row0, ROWS_PER_TILE)])


_INV_SQRT = float(1.0 / (1.0 + BN_EPS) ** 0.5)


def _mlp_body(x_ref, a0_ref, a1_ref, b_ref, w1_ref, b1_ref, g_ref, be_ref,
              w2_ref, b2_ref, o_ref, s_ref, c_ref, acc, cnt):
    i = pl.program_id(0)

    @pl.when(i == 0)
    def _init():
        acc[...] = jnp.zeros_like(acc)
        cnt[...] = jnp.zeros_like(cnt)

    z = x_ref[...] + a0_ref[0] + a1_ref[0]
    h = lax.dot_general(z, w1_ref[...], (((1,), (1,)), ((), ())),
                        preferred_element_type=jnp.float32)
    h = (h + b1_ref[...]) * (g_ref[...] * _INV_SQRT) + be_ref[...]
    h = jnp.maximum(h, 0.0)
    h = lax.dot_general(h, w2_ref[...], (((1,), (1,)), ((), ())),
                        preferred_element_type=jnp.float32)
    h = jnp.maximum(h + b2_ref[...], 0.0)
    o_ref[...] = h

    rows = lax.broadcasted_iota(jnp.int32, (BR, 1), 0) + i * BR
    valid = rows < N
    gids = lax.broadcasted_iota(jnp.int32, (BR, G), 1)
    onehot = jnp.where((b_ref[...] == gids) & valid, 1.0, 0.0)
    ones = jnp.where(jnp.broadcast_to(valid, (BR, D)), 1.0, 0.0)
    hm = jnp.where(jnp.broadcast_to(valid, (BR, D)), h, 0.0)

    cn = (((0,), (0,)), ((), ()))
    cnt[...] += lax.dot_general(onehot, ones, cn,
                                preferred_element_type=jnp.float32)
    acc[...] += lax.dot_general(onehot, hm, cn,
                                preferred_element_type=jnp.float32)

    @pl.when(i == GRID_R - 1)
    def _final():
        s_ref[...] = acc[...]
        c_ref[...] = cnt[...]


def _mlp(x, agg, batch2d, W1, b1, gamma, beta, W2, b2):
    return pl.pallas_call(
        _mlp_body,
        grid=(GRID_R,),
        in_specs=[
            pl.BlockSpec((BR, D), lambda i: (i, 0)),
            pl.BlockSpec((1, BR, D), lambda i: (0, i, 0)),
            pl.BlockSpec((1, BR, D), lambda i: (1, i, 0)),
            pl.BlockSpec((BR, 1), lambda i: (i, 0)),
            pl.BlockSpec((D, D), lambda i: (0, 0)),
            pl.BlockSpec((1, D), lambda i: (0, 0)),
            pl.BlockSpec((1, D), lambda i: (0, 0)),
            pl.BlockSpec((1, D), lambda i: (0, 0)),
            pl.BlockSpec((D, D), lambda i: (0, 0)),
            pl.BlockSpec((1, D), lambda i: (0, 0)),
        ],
        out_specs=[
            pl.BlockSpec((BR, D), lambda i: (i, 0)),
            pl.BlockSpec((G, D), lambda i: (0, 0)),
            pl.BlockSpec((G, D), lambda i: (0, 0)),
        ],
        out_shape=[
            jax.ShapeDtypeStruct((N, D), jnp.float32),
            jax.ShapeDtypeStruct((G, D), jnp.float32),
            jax.ShapeDtypeStruct((G, D), jnp.float32),
        ],
        scratch_shapes=[
            pltpu.VMEM((G, D), jnp.float32),
            pltpu.VMEM((G, D), jnp.float32),
        ],
        compiler_params=pltpu.CompilerParams(
            dimension_semantics=("arbitrary",)),
    )(x, agg, agg, batch2d, W1, b1.reshape(1, D), gamma.reshape(1, D),
      beta.reshape(1, D), W2, b2.reshape(1, D))


EB = E // CHUNK
EPB = E_PAD // CHUNK
PB = EPB - EB


def _prep_body(e_ref, s_ref, d_ref):
    s_ref[:EB] = e_ref[0]
    d_ref[:EB] = e_ref[1]
    ii = (lax.broadcasted_iota(jnp.int32, (PB, CHUNK), 0) * CHUNK +
          lax.broadcasted_iota(jnp.int32, (PB, CHUNK), 1))
    s_ref[EB:] = lax.rem(ii, N)
    d_ref[EB:] = N + lax.rem(ii, N_PAD - N)


def _prep(e32):
    return pl.pallas_call(
        _prep_body,
        out_shape=[
            jax.ShapeDtypeStruct((EPB, CHUNK), jnp.int32),
            jax.ShapeDtypeStruct((EPB, CHUNK), jnp.int32),
        ],
    )(e32)


def _head_body(s1_ref, s2_ref, s3_ref, c_ref, w1_ref, bb1_ref, w2_ref,
               bb2_ref, o_ref):
    c = jnp.maximum(c_ref[...], 1.0)
    pooled = jnp.concatenate(
        [s1_ref[...] / c, s2_ref[...] / c, s3_ref[...] / c], axis=1)
    hh = lax.dot_general(pooled, w1_ref[...], (((1,), (1,)), ((), ())),
                         preferred_element_type=jnp.float32)
    hh = jnp.maximum(hh + bb1_ref[...], 0.0)
    out = lax.dot_general(hh, w2_ref[...], (((1,), (1,)), ((), ())),
                          preferred_element_type=jnp.float32)
    o_ref[...] = out + bb2_ref[...]


def _head(s1, s2, s3, cnt, lin1_W, lin1_b, lin2_W, lin2_b):
    return pl.pallas_call(
        _head_body,
        out_shape=jax.ShapeDtypeStruct((G, 3), jnp.float32),
    )(s1, s2, s3, cnt, lin1_W, lin1_b.reshape(1, 3 * D), lin2_W,
      lin2_b.reshape(1, 3))


def kernel(x, edge_index, batch,
           conv1_W1, conv1_b1, conv1_gamma, conv1_beta, conv1_W2, conv1_b2,
           conv2_W1, conv2_b1, conv2_gamma, conv2_beta, conv2_W2, conv2_b2,
           conv3_W1, conv3_b1, conv3_gamma, conv3_beta, conv3_W2, conv3_b2,
           lin1_W, lin1_b, lin2_W, lin2_b):
    e32 = edge_index.astype(jnp.int32).reshape(2, EB, CHUNK)
    src2, dst2 = _prep(e32)
    src3 = src2.reshape(NW, CHUNKS_PER_TILE, CHUNK)
    dst3 = dst2.reshape(NW, CHUNKS_PER_TILE, CHUNK)
    zeros = jnp.zeros((NS * CHUNK, D), jnp.float32)
    batch2d = batch.astype(jnp.int32).reshape(N, 1)

    agg1 = _sc_scatter_add(x, src3, dst3, zeros)
    h1, s1, cnt = _mlp(x, agg1, batch2d, conv1_W1, conv1_b1, conv1_gamma,
                       conv1_beta, conv1_W2, conv1_b2)
    agg2 = _sc_scatter_add(h1, src3, dst3, zeros)
    h2, s2, _ = _mlp(h1, agg2, batch2d, conv2_W1, conv2_b1, conv2_gamma,
                     conv2_beta, conv2_W2, conv2_b2)
    agg3 = _sc_scatter_add(h2, src3, dst3, zeros)
    h3, s3, _ = _mlp(h2, agg3, batch2d, conv3_W1, conv3_b1, conv3_gamma,
                     conv3_beta, conv3_W2, conv3_b2)
    return _head(s1, s2, s3, cnt, lin1_W, lin1_b, lin2_W, lin2_b)


# speedup vs baseline: 11.6430x; 1.0172x over previous
import functools

import jax
import jax.numpy as jnp
from jax import lax
from jax.experimental import pallas as pl
from jax.experimental.pallas import tpu as pltpu
from jax.experimental.pallas import tpu_sc as plsc

N = 10000
E = 320000
D = 128
G = 64
BN_EPS = 1e-5

NC, NS = 2, 16
NW = NC * NS
CHUNK = 128
CHUNKS_PER_TILE = 80
HALF_CHUNKS = CHUNKS_PER_TILE // 2
EDGES_PER_TILE = CHUNKS_PER_TILE * CHUNK
HALF_EDGES = HALF_CHUNKS * CHUNK
LAST_REAL = E - (NW - 1) * EDGES_PER_TILE
E_PAD = NW * CHUNKS_PER_TILE * CHUNK
N_PADIDX = E_PAD - E
N_PAD = 10240
ROWS_PER_TILE = N_PAD // NS
ZCOPIES = ROWS_PER_TILE // CHUNK

BR = 5120
GRID_R = N_PAD // BR


_sc_mesh = plsc.VectorSubcoreMesh(
    core_axis_name="c", subcore_axis_name="s", num_cores=NC, num_subcores=NS)


@functools.partial(
    pl.kernel,
    out_type=jax.ShapeDtypeStruct((NC, N_PAD, D), jnp.float32),
    mesh=_sc_mesh,
    scratch_types=[
        pltpu.VMEM((HALF_EDGES,), jnp.int32),
        pltpu.VMEM((HALF_EDGES,), jnp.int32),
        pltpu.VMEM((CHUNK, D), jnp.float32),
        pltpu.VMEM((CHUNK, D), jnp.float32),
        pltpu.VMEM_SHARED((N_PAD, D), jnp.float32),
        pltpu.SemaphoreType.DMA,
        pltpu.SemaphoreType.DMA,
        pltpu.SemaphoreType.DMA,
    ],
)
def _sc_scatter_add(x_hbm, e_hbm, psrc_hbm, pdst_hbm, zero_hbm, out_hbm,
                    src_v, dst_v, rows0, rows1, acc_sh, sem0, sem1, semi):
    cid = lax.axis_index("c")
    sid = lax.axis_index("s")
    wid = cid * NS + sid
    row0 = sid * ROWS_PER_TILE
    base = wid * EDGES_PER_TILE
    last = wid == NW - 1
    PAD0 = HALF_EDGES - LAST_REAL

    def _stage(half, cp):
        @pl.when(jnp.logical_not(last))
        def _():
            cp(e_hbm.at[0, pl.ds(base + half * HALF_EDGES, HALF_EDGES)],
               src_v, semi)
            cp(e_hbm.at[1, pl.ds(base + half * HALF_EDGES, HALF_EDGES)],
               dst_v, semi)

        if half == 0:
            @pl.when(last)
            def _():
                cp(e_hbm.at[0, pl.ds(base, LAST_REAL)],
                   src_v.at[pl.ds(0, LAST_REAL)], semi)
                cp(psrc_hbm.at[pl.ds(0, PAD0)],
                   src_v.at[pl.ds(LAST_REAL, PAD0)], semi)
                cp(e_hbm.at[1, pl.ds(base, LAST_REAL)],
                   dst_v.at[pl.ds(0, LAST_REAL)], semi)
                cp(pdst_hbm.at[pl.ds(0, PAD0)],
                   dst_v.at[pl.ds(LAST_REAL, PAD0)], semi)
        else:
            @pl.when(last)
            def _():
                cp(psrc_hbm.at[pl.ds(PAD0, HALF_EDGES)], src_v, semi)
                cp(pdst_hbm.at[pl.ds(PAD0, HALF_EDGES)], dst_v, semi)

    def _wait(src, dst, sem):
        pltpu.make_async_copy(src, dst, sem).wait()

    _stage(0, pltpu.async_copy)

    pltpu.sync_copy(zero_hbm.at[pl.ds(sid * CHUNK, CHUNK)], rows0)
    for k in range(ZCOPIES):
        pltpu.sync_copy(rows0, acc_sh.at[pl.ds(row0 + k * CHUNK, CHUNK)])
    _stage(0, _wait)
    plsc.subcore_barrier()

    for half in range(2):
        if half:
            _stage(half, pltpu.async_copy)
            _stage(half, _wait)
        pltpu.async_copy(x_hbm.at[src_v.at[pl.ds(0, CHUNK)]], rows0, sem0)
        pltpu.async_copy(x_hbm.at[src_v.at[pl.ds(CHUNK, CHUNK)]], rows1, sem1)

        def body(p, carry):
            j0 = 2 * p * CHUNK
            pltpu.make_async_copy(x_hbm.at[src_v.at[pl.ds(0, CHUNK)]],
                                  rows0, sem0).wait()
            pltpu.sync_copy(rows0, acc_sh.at[dst_v.at[pl.ds(j0, CHUNK)]],
                            add=True)
            pltpu.async_copy(x_hbm.at[src_v.at[pl.ds(j0 + 2 * CHUNK, CHUNK)]],
                             rows0, sem0)
            pltpu.make_async_copy(x_hbm.at[src_v.at[pl.ds(0, CHUNK)]],
                                  rows1, sem1).wait()
            pltpu.sync_copy(rows1,
                            acc_sh.at[dst_v.at[pl.ds(j0 + CHUNK, CHUNK)]],
                            add=True)
            pltpu.async_copy(x_hbm.at[src_v.at[pl.ds(j0 + 3 * CHUNK, CHUNK)]],
                             rows1, sem1)
            return carry

        lax.fori_loop(0, HALF_CHUNKS // 2 - 1, body, 0)
        pltpu.make_async_copy(x_hbm.at[src_v.at[pl.ds(0, CHUNK)]],
                              rows0, sem0).wait()
        pltpu.sync_copy(
            rows0, acc_sh.at[dst_v.at[pl.ds(HALF_EDGES - 2 * CHUNK, CHUNK)]],
            add=True)
        pltpu.make_async_copy(x_hbm.at[src_v.at[pl.ds(0, CHUNK)]],
                              rows1, sem1).wait()
        pltpu.sync_copy(
            rows1, acc_sh.at[dst_v.at[pl.ds(HALF_EDGES - CHUNK, CHUNK)]],
            add=True)
    plsc.subcore_barrier()

    pltpu.sync_copy(acc_sh.at[pl.ds(row0, ROWS_PER_TILE)],
                    out_hbm.at[cid, pl.ds(row0, ROWS_PER_TILE)])


_INV_SQRT = float(1.0 / (1.0 + BN_EPS) ** 0.5)


def _mlp_body(x_ref, a0_ref, a1_ref, b_ref, w1_ref, b1_ref, g_ref, be_ref,
              w2_ref, b2_ref, o_ref, s_ref, c_ref, acc, cnt):
    i = pl.program_id(0)

    @pl.when(i == 0)
    def _init():
        acc[...] = jnp.zeros_like(acc)
        cnt[...] = jnp.zeros_like(cnt)

    z = x_ref[...] + a0_ref[0] + a1_ref[0]
    h = lax.dot_general(z, w1_ref[...], (((1,), (1,)), ((), ())),
                        preferred_element_type=jnp.float32)
    h = (h + b1_ref[...]) * (g_ref[...] * _INV_SQRT) + be_ref[...]
    h = jnp.maximum(h, 0.0)
    h = lax.dot_general(h, w2_ref[...], (((1,), (1,)), ((), ())),
                        preferred_element_type=jnp.float32)
    h = jnp.maximum(h + b2_ref[...], 0.0)
    o_ref[...] = h

    rows = lax.broadcasted_iota(jnp.int32, (BR, 1), 0) + i * BR
    valid = rows < N
    gids = lax.broadcasted_iota(jnp.int32, (BR, G), 1)
    onehot = jnp.where((b_ref[...] == gids) & valid, 1.0, 0.0)
    ones = jnp.where(jnp.broadcast_to(valid, (BR, D)), 1.0, 0.0)
    hm = jnp.where(jnp.broadcast_to(valid, (BR, D)), h, 0.0)

    cn = (((0,), (0,)), ((), ()))
    cnt[...] += lax.dot_general(onehot, ones, cn,
                                preferred_element_type=jnp.float32)
    acc[...] += lax.dot_general(onehot, hm, cn,
                                preferred_element_type=jnp.float32)

    @pl.when(i == GRID_R - 1)
    def _final():
        s_ref[...] = acc[...]
        c_ref[...] = cnt[...]


def _mlp(x, agg, batch2d, W1, b1, gamma, beta, W2, b2):
    return pl.pallas_call(
        _mlp_body,
        grid=(GRID_R,),
        in_specs=[
            pl.BlockSpec((BR, D), lambda i: (i, 0)),
            pl.BlockSpec((1, BR, D), lambda i: (0, i, 0)),
            pl.BlockSpec((1, BR, D), lambda i: (1, i, 0)),
            pl.BlockSpec((BR, 1), lambda i: (i, 0)),
            pl.BlockSpec((D, D), lambda i: (0, 0)),
            pl.BlockSpec((1, D), lambda i: (0, 0)),
            pl.BlockSpec((1, D), lambda i: (0, 0)),
            pl.BlockSpec((1, D), lambda i: (0, 0)),
            pl.BlockSpec((D, D), lambda i: (0, 0)),
            pl.BlockSpec((1, D), lambda i: (0, 0)),
        ],
        out_specs=[
            pl.BlockSpec((BR, D), lambda i: (i, 0)),
            pl.BlockSpec((G, D), lambda i: (0, 0)),
            pl.BlockSpec((G, D), lambda i: (0, 0)),
        ],
        out_shape=[
            jax.ShapeDtypeStruct((N, D), jnp.float32),
            jax.ShapeDtypeStruct((G, D), jnp.float32),
            jax.ShapeDtypeStruct((G, D), jnp.float32),
        ],
        scratch_shapes=[
            pltpu.VMEM((G, D), jnp.float32),
            pltpu.VMEM((G, D), jnp.float32),
        ],
        compiler_params=pltpu.CompilerParams(
            dimension_semantics=("arbitrary",)),
    )(x, agg, agg, batch2d, W1, b1.reshape(1, D), gamma.reshape(1, D),
      beta.reshape(1, D), W2, b2.reshape(1, D))


def _head_body(s1_ref, s2_ref, s3_ref, c_ref, w1_ref, bb1_ref, w2_ref,
               bb2_ref, o_ref):
    c = jnp.maximum(c_ref[...], 1.0)
    pooled = jnp.concatenate(
        [s1_ref[...] / c, s2_ref[...] / c, s3_ref[...] / c], axis=1)
    hh = lax.dot_general(pooled, w1_ref[...], (((1,), (1,)), ((), ())),
                         preferred_element_type=jnp.float32)
    hh = jnp.maximum(hh + bb1_ref[...], 0.0)
    out = lax.dot_general(hh, w2_ref[...], (((1,), (1,)), ((), ())),
                          preferred_element_type=jnp.float32)
    o_ref[...] = out + bb2_ref[...]


def _head(s1, s2, s3, cnt, lin1_W, lin1_b, lin2_W, lin2_b):
    return pl.pallas_call(
        _head_body,
        out_shape=jax.ShapeDtypeStruct((G, 3), jnp.float32),
    )(s1, s2, s3, cnt, lin1_W, lin1_b.reshape(1, 3 * D), lin2_W,
      lin2_b.reshape(1, 3))


def kernel(x, edge_index, batch,
           conv1_W1, conv1_b1, conv1_gamma, conv1_beta, conv1_W2, conv1_b2,
           conv2_W1, conv2_b1, conv2_gamma, conv2_beta, conv2_W2, conv2_b2,
           conv3_W1, conv3_b1, conv3_gamma, conv3_beta, conv3_W2, conv3_b2,
           lin1_W, lin1_b, lin2_W, lin2_b):
    e32 = edge_index.astype(jnp.int32)
    psrc = jnp.arange(N_PADIDX, dtype=jnp.int32) % N
    pdst = N + jnp.arange(N_PADIDX, dtype=jnp.int32) % (N_PAD - N)
    zeros = jnp.zeros((NS * CHUNK, D), jnp.float32)
    batch2d = batch.astype(jnp.int32).reshape(N, 1)

    agg1 = _sc_scatter_add(x, e32, psrc, pdst, zeros)
    h1, s1, cnt = _mlp(x, agg1, batch2d, conv1_W1, conv1_b1, conv1_gamma,
                       conv1_beta, conv1_W2, conv1_b2)
    agg2 = _sc_scatter_add(h1, e32, psrc, pdst, zeros)
    h2, s2, _ = _mlp(h1, agg2, batch2d, conv2_W1, conv2_b1, conv2_gamma,
                     conv2_beta, conv2_W2, conv2_b2)
    agg3 = _sc_scatter_add(h2, e32, psrc, pdst, zeros)
    h3, s3, _ = _mlp(h2, agg3, batch2d, conv3_W1, conv3_b1, conv3_gamma,
                     conv3_beta, conv3_W2, conv3_b2)
    return _head(s1, s2, s3, cnt, lin1_W, lin1_b, lin2_W, lin2_b)
